# baseline (speedup 1.0000x reference)
import functools

import jax
import jax.numpy as jnp
from jax import lax
from jax.experimental import pallas as pl
from jax.experimental.pallas import tpu as pltpu
from jax.experimental.pallas import tpu_sc as plsc

_N = 10000
_NPAD = 10240
_M = 320000
_D = 128
_NC = 2
_NS = 16
_L = 16
_B = 80
_EPT = _M // (_NC * _NS)
_NCHUNK = _EPT // _B
_RPT = _NPAD // _NS

_GDN = jax.lax.GatherDimensionNumbers(
    offset_dims=(), collapsed_slice_dims=(0,), start_index_map=(0,))


def _lane_bcast(v16, j):
  idx = jnp.full((_L,), j, jnp.int32)
  return jax.lax.gather(v16, idx[:, None], _GDN, slice_sizes=(1,),
                        mode=jax.lax.GatherScatterMode.PROMISE_IN_BOUNDS)


def _bf16_round(v):
  c = v * 65537.0
  return c - (c - v)


def _scale_rows(buf, w_v, bf16_round=False):
  for g in range(_B // _L):
    w16 = w_v[pl.ds(g * _L, _L)]
    for j in range(_L):
      r = g * _L + j
      wb = _lane_bcast(w16, j)
      for k in range(_D // _L):
        sl = pl.ds(k * _L, _L)
        x = buf[r, sl]
        if bf16_round:
          x = _bf16_round(x)
        buf[r, sl] = x * wb


def _sc_body_c(tf_hbm, tb_hbm, heads_hbm, tails_hbm, wf_hbm, wb_hbm,
               zeros_hbm, c_out,
               h0, t0, f0, b0, h1, t1, f1, b1,
               tf0, tb0, tf1, tb1,
               acc, si0, si1, sg0, sg1):
  c = lax.axis_index("c")
  s = lax.axis_index("s")

  r0 = s * _RPT
  pltpu.sync_copy(zeros_hbm, acc.at[pl.ds(r0, _RPT)])
  plsc.subcore_barrier()

  tile_base = (c * _NS + s) * _EPT
  bufs = ((h0, t0, f0, b0, tf0, tb0, si0, sg0),
          (h1, t1, f1, b1, tf1, tb1, si1, sg1))

  def idx_args(ci, bs):
    h, t, f, b, _, _, si, _ = bs
    base = tile_base + ci * _B
    return ((heads_hbm.at[pl.ds(base, _B)], h, si),
            (tails_hbm.at[pl.ds(base, _B)], t, si),
            (wf_hbm.at[pl.ds(base, _B)], f, si),
            (wb_hbm.at[pl.ds(base, _B)], b, si))

  def idx_start(ci, bs):
    for a in idx_args(ci, bs):
      pltpu.async_copy(*a)

  def idx_wait(ci, bs):
    for a in idx_args(ci, bs):
      pltpu.make_async_copy(*a).wait()

  def gather_start(bs):
    h, t, _, _, tf, tb, _, sg = bs
    pltpu.async_copy(tf_hbm.at[h], tf, sg)
    pltpu.async_copy(tb_hbm.at[t], tb, sg)

  def gather_wait(bs):
    h, t, _, _, tf, tb, _, sg = bs
    pltpu.make_async_copy(tf_hbm.at[h], tf, sg).wait()
    pltpu.make_async_copy(tb_hbm.at[t], tb, sg).wait()

  idx_start(0, bufs[0])
  idx_wait(0, bufs[0])
  gather_start(bufs[0])
  idx_start(1, bufs[1])

  def step(ci, cur, nxt):
    @pl.when(ci < _NCHUNK)
    def _():
      gather_wait(cur)

      @pl.when(ci + 1 < _NCHUNK)
      def _():
        idx_wait(ci + 1, nxt)
        gather_start(nxt)

      h, t, f, b, tf, tb, _, _ = cur
      _scale_rows(tf, f)
      _scale_rows(tb, b)
      pltpu.sync_copy(tf, acc.at[t], add=True)
      pltpu.sync_copy(tb, acc.at[h], add=True)

      @pl.when(ci + 2 < _NCHUNK)
      def _():
        idx_start(ci + 2, cur)

  def pair(p, carry):
    step(2 * p, bufs[0], bufs[1])
    step(2 * p + 1, bufs[1], bufs[0])
    return carry

  lax.fori_loop(0, (_NCHUNK + 1) // 2, pair, 0)
  plsc.subcore_barrier()
  pltpu.sync_copy(acc.at[pl.ds(r0, _RPT)], c_out.at[c, pl.ds(r0, _RPT)])


def _sc_body_q(e_hbm, dst_hbm, w_hbm, zeros_hbm, q_out,
               d0, w0, e0, d1, w1, e1, acc, s0, s1):
  c = lax.axis_index("c")
  s = lax.axis_index("s")

  r0 = s * _RPT
  pltpu.sync_copy(zeros_hbm, acc.at[pl.ds(r0, _RPT)])
  plsc.subcore_barrier()

  tile_base = (c * _NS + s) * _EPT
  bufs = ((d0, w0, e0, s0), (d1, w1, e1, s1))

  def in_args(ci, bs):
    d, w, e, sem = bs
    base = tile_base + ci * _B
    return ((dst_hbm.at[pl.ds(base, _B)], d, sem),
            (w_hbm.at[pl.ds(base, _B)], w, sem),
            (e_hbm.at[pl.ds(base, _B)], e, sem))

  def in_start(ci, bs):
    for a in in_args(ci, bs):
      pltpu.async_copy(*a)

  def in_wait(ci, bs):
    for a in in_args(ci, bs):
      pltpu.make_async_copy(*a).wait()

  in_start(0, bufs[0])
  in_start(1, bufs[1])

  def step(ci, cur):
    @pl.when(ci < _NCHUNK)
    def _():
      in_wait(ci, cur)
      d, w, e, _ = cur
      _scale_rows(e, w, bf16_round=True)
      pltpu.sync_copy(e, acc.at[d], add=True)

      @pl.when(ci + 2 < _NCHUNK)
      def _():
        in_start(ci + 2, cur)

  def pair(p, carry):
    step(2 * p, bufs[0])
    step(2 * p + 1, bufs[1])
    return carry

  lax.fori_loop(0, (_NCHUNK + 1) // 2, pair, 0)
  plsc.subcore_barrier()
  pltpu.sync_copy(acc.at[pl.ds(r0, _RPT)], q_out.at[c, pl.ds(r0, _RPT)])


def _sc_mesh():
  return plsc.VectorSubcoreMesh(core_axis_name="c", subcore_axis_name="s",
                                num_cores=_NC, num_subcores=_NS)


_OUT2 = jax.ShapeDtypeStruct((_NC, _NPAD, _D), jnp.float32)


@jax.jit
def _sc_accumulate(tf, tb, e, heads, tails, wf, wb, zeros):
  f_c = pl.kernel(
      _sc_body_c,
      out_type=_OUT2,
      mesh=_sc_mesh(),
      scratch_types=[
          pltpu.VMEM((_B,), jnp.int32),
          pltpu.VMEM((_B,), jnp.int32),
          pltpu.VMEM((_B,), jnp.float32),
          pltpu.VMEM((_B,), jnp.float32),
          pltpu.VMEM((_B,), jnp.int32),
          pltpu.VMEM((_B,), jnp.int32),
          pltpu.VMEM((_B,), jnp.float32),
          pltpu.VMEM((_B,), jnp.float32),
          pltpu.VMEM((_B, _D), jnp.float32),
          pltpu.VMEM((_B, _D), jnp.float32),
          pltpu.VMEM((_B, _D), jnp.float32),
          pltpu.VMEM((_B, _D), jnp.float32),
          pltpu.VMEM_SHARED((_NPAD, _D), jnp.float32),
          pltpu.SemaphoreType.DMA,
          pltpu.SemaphoreType.DMA,
          pltpu.SemaphoreType.DMA,
          pltpu.SemaphoreType.DMA,
      ],
  )
  f_q = pl.kernel(
      _sc_body_q,
      out_type=_OUT2,
      mesh=_sc_mesh(),
      scratch_types=[
          pltpu.VMEM((_B,), jnp.int32),
          pltpu.VMEM((_B,), jnp.float32),
          pltpu.VMEM((_B, _D), jnp.float32),
          pltpu.VMEM((_B,), jnp.int32),
          pltpu.VMEM((_B,), jnp.float32),
          pltpu.VMEM((_B, _D), jnp.float32),
          pltpu.VMEM_SHARED((_NPAD, _D), jnp.float32),
          pltpu.SemaphoreType.DMA,
          pltpu.SemaphoreType.DMA,
      ],
  )
  C2 = f_c(tf, tb, heads, tails, wf, wb, zeros)
  Qf2 = f_q(e, tails, wf, zeros)
  Qb2 = f_q(e, heads, wb, zeros)
  return C2, Qf2, Qb2


def _pre_body(h_ref, wf_ref, bf_ref, wb_ref, bb_ref, tf_ref, tb_ref):
  h = h_ref[...]
  tf_ref[...] = lax.dot_general(h, wf_ref[...][:, :_D],
                                (((1,), (1,)), ((), ()))) + bf_ref[...]
  tb_ref[...] = lax.dot_general(h, wb_ref[...][:, :_D],
                                (((1,), (1,)), ((), ()))) + bb_ref[...]


@jax.jit
def _pre_tables(H, W_fwd, b_fwd, W_back, b_back):
  blk = 1000
  grid = (_N // blk,)
  return pl.pallas_call(
      _pre_body,
      grid=grid,
      in_specs=[
          pl.BlockSpec((blk, _D), lambda i: (i, 0)),
          pl.BlockSpec((_D, 2 * _D), lambda i: (0, 0)),
          pl.BlockSpec((1, _D), lambda i: (0, 0)),
          pl.BlockSpec((_D, 2 * _D), lambda i: (0, 0)),
          pl.BlockSpec((1, _D), lambda i: (0, 0)),
      ],
      out_specs=[
          pl.BlockSpec((blk, _D), lambda i: (i, 0)),
          pl.BlockSpec((blk, _D), lambda i: (i, 0)),
      ],
      out_shape=[jax.ShapeDtypeStruct((_N, _D), jnp.float32)] * 2,
  )(H, W_fwd, b_fwd, W_back, b_back)


def _post_body(c0_ref, c1_ref, qf0_ref, qf1_ref, qb0_ref, qb1_ref, h_ref,
               wf_ref, wb_ref, g_ref, b_ref, o_ref):
  agg = c0_ref[0] + c1_ref[0]
  qf = qf0_ref[0] + qf1_ref[0]
  qb = qb0_ref[0] + qb1_ref[0]
  w2f = wf_ref[...][:, _D:].astype(jnp.bfloat16).astype(jnp.float32)
  w2b = wb_ref[...][:, _D:].astype(jnp.bfloat16).astype(jnp.float32)
  agg = agg + lax.dot_general(qf, w2f, (((1,), (1,)), ((), ())),
                              precision=jax.lax.Precision.HIGHEST)
  agg = agg + lax.dot_general(qb, w2b, (((1,), (1,)), ((), ())),
                              precision=jax.lax.Precision.HIGHEST)
  agg = agg / jnp.sum(agg, axis=1, keepdims=True)
  x = jnp.where(agg >= 0, agg, 0.01 * agg) + h_ref[...]
  mean = jnp.mean(x, axis=1, keepdims=True)
  xc = x - mean
  var = jnp.mean(xc * xc, axis=1, keepdims=True)
  o_ref[...] = xc * jax.lax.rsqrt(var + 1e-5) * g_ref[...] + b_ref[...]


@jax.jit
def _post_combine(C2, Qf2, Qb2, H, W_fwd, W_back, gamma, beta):
  blk = 1000
  grid = (_N // blk,)
  spec0 = pl.BlockSpec((1, blk, _D), lambda i: (0, i, 0))
  spec1 = pl.BlockSpec((1, blk, _D), lambda i: (1, i, 0))
  rspec = pl.BlockSpec((blk, _D), lambda i: (i, 0))
  wspec = pl.BlockSpec((_D, 2 * _D), lambda i: (0, 0))
  vspec = pl.BlockSpec((1, _D), lambda i: (0, 0))
  return pl.pallas_call(
      _post_body,
      grid=grid,
      in_specs=[spec0, spec1, spec0, spec1, spec0, spec1, rspec,
                wspec, wspec, vspec, vspec],
      out_specs=rspec,
      out_shape=jax.ShapeDtypeStruct((_N, _D), jnp.float32),
  )(C2, C2, Qf2, Qf2, Qb2, Qb2, H, W_fwd, W_back, gamma, beta)


def kernel(H, E, ht, queries, influence_weights, W_fwd, b_fwd, W_back, b_back,
           ln_gamma, ln_beta):
  del queries
  heads = ht[:, 0]
  tails = ht[:, 1]
  wf = influence_weights[:_M, 0]
  wb = influence_weights[_M:, 0]

  Tf, Tb = _pre_tables(H, W_fwd, b_fwd.reshape(1, _D), W_back,
                       b_back.reshape(1, _D))
  zeros = jnp.zeros((_RPT, _D), jnp.float32)

  C2, Qf2, Qb2 = _sc_accumulate(Tf, Tb, E, heads, tails, wf, wb, zeros)

  return _post_combine(C2, Qf2, Qb2, H, W_fwd, W_back,
                       ln_gamma.reshape(1, _D), ln_beta.reshape(1, _D))


# speedup vs baseline: 1.0831x; 1.0831x over previous
import functools

import jax
import jax.numpy as jnp
from jax import lax
from jax.experimental import pallas as pl
from jax.experimental.pallas import tpu as pltpu
from jax.experimental.pallas import tpu_sc as plsc

_N = 10000
_NPAD = 10240
_M = 320000
_D = 128
_NC = 2
_NS = 16
_L = 16
_B = 80
_EPT = _M // (_NC * _NS)
_NCHUNK = _EPT // _B
_RPT = _NPAD // _NS

_GDN = jax.lax.GatherDimensionNumbers(
    offset_dims=(), collapsed_slice_dims=(0,), start_index_map=(0,))


def _lane_bcast(v16, j):
  idx = jnp.full((_L,), j, jnp.int32)
  return jax.lax.gather(v16, idx[:, None], _GDN, slice_sizes=(1,),
                        mode=jax.lax.GatherScatterMode.PROMISE_IN_BOUNDS)


def _bf16_round(v):
  c = v * 65537.0
  return c - (c - v)


def _scale_rows(buf, w_v, bf16_round=False):
  for g in range(_B // _L):
    w16 = w_v[pl.ds(g * _L, _L)]
    for j in range(_L):
      r = g * _L + j
      wb = _lane_bcast(w16, j)
      for k in range(_D // _L):
        sl = pl.ds(k * _L, _L)
        x = buf[r, sl]
        if bf16_round:
          x = _bf16_round(x)
        buf[r, sl] = x * wb


def _copy_idx(src, dst):
  for g in range(_B // _L):
    sl = pl.ds(g * _L, _L)
    dst[sl] = src[sl]


def _sc_body_c(tf_hbm, tb_hbm, heads_hbm, tails_hbm, wf_hbm, wb_hbm,
               zeros_hbm, c_out,
               h0, t0, f0, b0, sh0, st0, h1, t1, f1, b1, sh1, st1,
               tf0, tb0, tf1, tb1,
               acc, si0, si1, sg0, sg1, ss0, ss1):
  c = lax.axis_index("c")
  s = lax.axis_index("s")

  r0 = s * _RPT
  pltpu.sync_copy(zeros_hbm, acc.at[pl.ds(r0, _RPT)])
  plsc.subcore_barrier()

  tile_base = (c * _NS + s) * _EPT
  bufs = ((h0, t0, f0, b0, sh0, st0, tf0, tb0, si0, sg0, ss0),
          (h1, t1, f1, b1, sh1, st1, tf1, tb1, si1, sg1, ss1))

  def idx_args(ci, bs):
    h, t, f, b = bs[0], bs[1], bs[2], bs[3]
    si = bs[8]
    base = tile_base + ci * _B
    return ((heads_hbm.at[pl.ds(base, _B)], h, si),
            (tails_hbm.at[pl.ds(base, _B)], t, si),
            (wf_hbm.at[pl.ds(base, _B)], f, si),
            (wb_hbm.at[pl.ds(base, _B)], b, si))

  def idx_start(ci, bs):
    for a in idx_args(ci, bs):
      pltpu.async_copy(*a)

  def idx_wait(ci, bs):
    for a in idx_args(ci, bs):
      pltpu.make_async_copy(*a).wait()

  def gather_start(bs):
    h, t, tf, tb, sg = bs[0], bs[1], bs[6], bs[7], bs[9]
    pltpu.async_copy(tf_hbm.at[h], tf, sg)
    pltpu.async_copy(tb_hbm.at[t], tb, sg)

  def gather_wait(bs):
    h, t, tf, tb, sg = bs[0], bs[1], bs[6], bs[7], bs[9]
    pltpu.make_async_copy(tf_hbm.at[h], tf, sg).wait()
    pltpu.make_async_copy(tb_hbm.at[t], tb, sg).wait()

  def scatter_start(bs):
    sh, st, tf, tb, ss = bs[4], bs[5], bs[6], bs[7], bs[10]
    pltpu.async_copy(tf, acc.at[st], ss, add=True)
    pltpu.async_copy(tb, acc.at[sh], ss, add=True)

  def scatter_wait(bs):
    sh, st, tf, tb, ss = bs[4], bs[5], bs[6], bs[7], bs[10]
    pltpu.make_async_copy(tf, acc.at[st], ss).wait()
    pltpu.make_async_copy(tb, acc.at[sh], ss).wait()

  idx_start(0, bufs[0])
  idx_wait(0, bufs[0])
  gather_start(bufs[0])
  idx_start(1, bufs[1])

  def step(ci, cur, nxt):
    @pl.when(ci < _NCHUNK)
    def _():
      gather_wait(cur)

      @pl.when(ci + 1 < _NCHUNK)
      def _():
        @pl.when(ci >= 1)
        def _():
          scatter_wait(nxt)
        idx_wait(ci + 1, nxt)
        gather_start(nxt)

      h, t, f, b, sh, st, tf, tb = (cur[0], cur[1], cur[2], cur[3], cur[4],
                                    cur[5], cur[6], cur[7])
      _scale_rows(tf, f)
      _scale_rows(tb, b)
      _copy_idx(h, sh)
      _copy_idx(t, st)
      scatter_start(cur)

      @pl.when(ci + 2 < _NCHUNK)
      def _():
        idx_start(ci + 2, cur)

  def pair(p, carry):
    step(2 * p, bufs[0], bufs[1])
    step(2 * p + 1, bufs[1], bufs[0])
    return carry

  lax.fori_loop(0, (_NCHUNK + 1) // 2, pair, 0)
  scatter_wait(bufs[(_NCHUNK - 2) % 2])
  scatter_wait(bufs[(_NCHUNK - 1) % 2])
  plsc.subcore_barrier()
  pltpu.sync_copy(acc.at[pl.ds(r0, _RPT)], c_out.at[c, pl.ds(r0, _RPT)])


def _sc_body_q(e_hbm, dst_hbm, w_hbm, zeros_hbm, q_out,
               d0, w0, sd0, e0, d1, w1, sd1, e1, d2, w2, sd2, e2,
               acc, si0, si1, si2, ss0, ss1, ss2):
  c = lax.axis_index("c")
  s = lax.axis_index("s")

  r0 = s * _RPT
  pltpu.sync_copy(zeros_hbm, acc.at[pl.ds(r0, _RPT)])
  plsc.subcore_barrier()

  tile_base = (c * _NS + s) * _EPT
  bufs = ((d0, w0, sd0, e0, si0, ss0),
          (d1, w1, sd1, e1, si1, ss1),
          (d2, w2, sd2, e2, si2, ss2))

  def in_args(ci, bs):
    d, w, _, e, si, _ = bs
    base = tile_base + ci * _B
    return ((dst_hbm.at[pl.ds(base, _B)], d, si),
            (w_hbm.at[pl.ds(base, _B)], w, si),
            (e_hbm.at[pl.ds(base, _B)], e, si))

  def in_start(ci, bs):
    for a in in_args(ci, bs):
      pltpu.async_copy(*a)

  def in_wait(ci, bs):
    for a in in_args(ci, bs):
      pltpu.make_async_copy(*a).wait()

  def scatter_start(bs):
    _, _, sd, e, _, ss = bs
    pltpu.async_copy(e, acc.at[sd], ss, add=True)

  def scatter_wait(bs):
    _, _, sd, e, _, ss = bs
    pltpu.make_async_copy(e, acc.at[sd], ss).wait()

  in_start(0, bufs[0])
  in_start(1, bufs[1])
  in_start(2, bufs[2])

  def step(ci, cur, prv):
    @pl.when(ci < _NCHUNK)
    def _():
      in_wait(ci, cur)

      @pl.when(jnp.logical_and(ci >= 1, ci + 2 < _NCHUNK))
      def _():
        scatter_wait(prv)
        in_start(ci + 2, prv)

      d, w, sd, e = cur[0], cur[1], cur[2], cur[3]
      _scale_rows(e, w, bf16_round=True)
      _copy_idx(d, sd)
      scatter_start(cur)

  def trio(p, carry):
    step(3 * p, bufs[0], bufs[2])
    step(3 * p + 1, bufs[1], bufs[0])
    step(3 * p + 2, bufs[2], bufs[1])
    return carry

  lax.fori_loop(0, (_NCHUNK + 2) // 3, trio, 0)
  scatter_wait(bufs[(_NCHUNK - 3) % 3])
  scatter_wait(bufs[(_NCHUNK - 2) % 3])
  scatter_wait(bufs[(_NCHUNK - 1) % 3])
  plsc.subcore_barrier()
  pltpu.sync_copy(acc.at[pl.ds(r0, _RPT)], q_out.at[c, pl.ds(r0, _RPT)])


def _sc_mesh():
  return plsc.VectorSubcoreMesh(core_axis_name="c", subcore_axis_name="s",
                                num_cores=_NC, num_subcores=_NS)


_OUT2 = jax.ShapeDtypeStruct((_NC, _NPAD, _D), jnp.float32)


@jax.jit
def _sc_accumulate(tf, tb, e, heads, tails, wf, wb, zeros):
  f_c = pl.kernel(
      _sc_body_c,
      out_type=_OUT2,
      mesh=_sc_mesh(),
      scratch_types=(
          [pltpu.VMEM((_B,), jnp.int32)] * 2 +
          [pltpu.VMEM((_B,), jnp.float32)] * 2 +
          [pltpu.VMEM((_B,), jnp.int32)] * 2 +
          [pltpu.VMEM((_B,), jnp.int32)] * 2 +
          [pltpu.VMEM((_B,), jnp.float32)] * 2 +
          [pltpu.VMEM((_B,), jnp.int32)] * 2 +
          [pltpu.VMEM((_B, _D), jnp.float32)] * 4 +
          [pltpu.VMEM_SHARED((_NPAD, _D), jnp.float32)] +
          [pltpu.SemaphoreType.DMA] * 6
      ),
  )
  f_q = pl.kernel(
      _sc_body_q,
      out_type=_OUT2,
      mesh=_sc_mesh(),
      scratch_types=(
          ([pltpu.VMEM((_B,), jnp.int32),
            pltpu.VMEM((_B,), jnp.float32),
            pltpu.VMEM((_B,), jnp.int32),
            pltpu.VMEM((_B, _D), jnp.float32)
            ] * 3) +
          [pltpu.VMEM_SHARED((_NPAD, _D), jnp.float32)] +
          [pltpu.SemaphoreType.DMA] * 6
      ),
  )
  C2 = f_c(tf, tb, heads, tails, wf, wb, zeros)
  Qf2 = f_q(e, tails, wf, zeros)
  Qb2 = f_q(e, heads, wb, zeros)
  return C2, Qf2, Qb2


def _pre_body(h_ref, wf_ref, bf_ref, wb_ref, bb_ref, tf_ref, tb_ref):
  h = h_ref[...]
  tf_ref[...] = lax.dot_general(h, wf_ref[...][:, :_D],
                                (((1,), (1,)), ((), ()))) + bf_ref[...]
  tb_ref[...] = lax.dot_general(h, wb_ref[...][:, :_D],
                                (((1,), (1,)), ((), ()))) + bb_ref[...]


@jax.jit
def _pre_tables(H, W_fwd, b_fwd, W_back, b_back):
  blk = 1000
  grid = (_N // blk,)
  return pl.pallas_call(
      _pre_body,
      grid=grid,
      in_specs=[
          pl.BlockSpec((blk, _D), lambda i: (i, 0)),
          pl.BlockSpec((_D, 2 * _D), lambda i: (0, 0)),
          pl.BlockSpec((1, _D), lambda i: (0, 0)),
          pl.BlockSpec((_D, 2 * _D), lambda i: (0, 0)),
          pl.BlockSpec((1, _D), lambda i: (0, 0)),
      ],
      out_specs=[
          pl.BlockSpec((blk, _D), lambda i: (i, 0)),
          pl.BlockSpec((blk, _D), lambda i: (i, 0)),
      ],
      out_shape=[jax.ShapeDtypeStruct((_N, _D), jnp.float32)] * 2,
  )(H, W_fwd, b_fwd, W_back, b_back)


def _post_body(c0_ref, c1_ref, qf0_ref, qf1_ref, qb0_ref, qb1_ref, h_ref,
               wf_ref, wb_ref, g_ref, b_ref, o_ref):
  agg = c0_ref[0] + c1_ref[0]
  qf = qf0_ref[0] + qf1_ref[0]
  qb = qb0_ref[0] + qb1_ref[0]
  w2f = wf_ref[...][:, _D:].astype(jnp.bfloat16).astype(jnp.float32)
  w2b = wb_ref[...][:, _D:].astype(jnp.bfloat16).astype(jnp.float32)
  agg = agg + lax.dot_general(qf, w2f, (((1,), (1,)), ((), ())),
                              precision=jax.lax.Precision.HIGHEST)
  agg = agg + lax.dot_general(qb, w2b, (((1,), (1,)), ((), ())),
                              precision=jax.lax.Precision.HIGHEST)
  agg = agg / jnp.sum(agg, axis=1, keepdims=True)
  x = jnp.where(agg >= 0, agg, 0.01 * agg) + h_ref[...]
  mean = jnp.mean(x, axis=1, keepdims=True)
  xc = x - mean
  var = jnp.mean(xc * xc, axis=1, keepdims=True)
  o_ref[...] = xc * jax.lax.rsqrt(var + 1e-5) * g_ref[...] + b_ref[...]


@jax.jit
def _post_combine(C2, Qf2, Qb2, H, W_fwd, W_back, gamma, beta):
  blk = 1000
  grid = (_N // blk,)
  spec0 = pl.BlockSpec((1, blk, _D), lambda i: (0, i, 0))
  spec1 = pl.BlockSpec((1, blk, _D), lambda i: (1, i, 0))
  rspec = pl.BlockSpec((blk, _D), lambda i: (i, 0))
  wspec = pl.BlockSpec((_D, 2 * _D), lambda i: (0, 0))
  vspec = pl.BlockSpec((1, _D), lambda i: (0, 0))
  return pl.pallas_call(
      _post_body,
      grid=grid,
      in_specs=[spec0, spec1, spec0, spec1, spec0, spec1, rspec,
                wspec, wspec, vspec, vspec],
      out_specs=rspec,
      out_shape=jax.ShapeDtypeStruct((_N, _D), jnp.float32),
  )(C2, C2, Qf2, Qf2, Qb2, Qb2, H, W_fwd, W_back, gamma, beta)


def kernel(H, E, ht, queries, influence_weights, W_fwd, b_fwd, W_back, b_back,
           ln_gamma, ln_beta):
  del queries
  heads = ht[:, 0]
  tails = ht[:, 1]
  wf = influence_weights[:_M, 0]
  wb = influence_weights[_M:, 0]

  Tf, Tb = _pre_tables(H, W_fwd, b_fwd.reshape(1, _D), W_back,
                       b_back.reshape(1, _D))
  zeros = jnp.zeros((_RPT, _D), jnp.float32)

  C2, Qf2, Qb2 = _sc_accumulate(Tf, Tb, E, heads, tails, wf, wb, zeros)

  return _post_combine(C2, Qf2, Qb2, H, W_fwd, W_back,
                       ln_gamma.reshape(1, _D), ln_beta.reshape(1, _D))


# speedup vs baseline: 1.7162x; 1.5846x over previous
import functools

import jax
import jax.numpy as jnp
from jax import lax
from jax.experimental import pallas as pl
from jax.experimental.pallas import tpu as pltpu
from jax.experimental.pallas import tpu_sc as plsc

_N = 10000
_NPAD = 10112
_M = 320000
_D = 128
_NC = 2
_NS = 16
_NW = _NC * _NS
_L = 16
_BC = 80
_EPT = _M // _NW
_NCC = _EPT // _BC
_BQ = 128
_NCG = _M // _BQ
_ITER = -(-_NCG // _NW)
_RPT = _NPAD // _NS

_GDN = jax.lax.GatherDimensionNumbers(
    offset_dims=(), collapsed_slice_dims=(0,), start_index_map=(0,))


def _lane_bcast(v16, j):
  idx = jnp.full((_L,), j, jnp.int32)
  return jax.lax.gather(v16, idx[:, None], _GDN, slice_sizes=(1,),
                        mode=jax.lax.GatherScatterMode.PROMISE_IN_BOUNDS)


def _bf16_round(v):
  c = v * 65537.0
  return c - (c - v)


def _scale_group(buf, w_v, g, bf16_round):
  w16 = w_v[pl.ds(g * _L, _L)]
  for j in range(_L):
    r = g * _L + j
    wb = _lane_bcast(w16, j)
    for k in range(_D // _L):
      sl = pl.ds(k * _L, _L)
      x = buf[r, sl]
      if bf16_round:
        x = _bf16_round(x)
      buf[r, sl] = x * wb


def _scale_rows(buf, w_v, nb, bf16_round=False):
  def body(g, carry):
    _scale_group(buf, w_v, g, bf16_round)
    return carry
  lax.fori_loop(0, nb // _L, body, 0)


def _scale_rows2(buf_a, w_a, buf_b, w_b, nb):
  def body(g, carry):
    _scale_group(buf_a, w_a, g, False)
    _scale_group(buf_b, w_b, g, False)
    return carry
  lax.fori_loop(0, nb // _L, body, 0)


def _copy_idx(src, dst, nb):
  for g in range(nb // _L):
    sl = pl.ds(g * _L, _L)
    dst[sl] = src[sl]


def _sc_body_c(tf_hbm, tb_hbm, heads_hbm, tails_hbm, wf_hbm, wb_hbm,
               zeros_hbm, c_out,
               h0, t0, f0, b0, sh0, st0, h1, t1, f1, b1, sh1, st1,
               tf0, tb0, tf1, tb1,
               acc, si0, si1, sg0, sg1, ss0, ss1):
  c = lax.axis_index("c")
  s = lax.axis_index("s")
  wid = c * _NS + s

  r0 = s * _RPT
  pltpu.sync_copy(zeros_hbm, acc.at[pl.ds(r0, _RPT)])
  plsc.subcore_barrier()

  bufs = ((h0, t0, f0, b0, sh0, st0, tf0, tb0, si0, sg0, ss0),
          (h1, t1, f1, b1, sh1, st1, tf1, tb1, si1, sg1, ss1))

  tile_base = wid * _EPT

  def idx_args(ci, bs):
    h, t, f, b = bs[0], bs[1], bs[2], bs[3]
    si = bs[8]
    base = tile_base + ci * _BC
    return ((heads_hbm.at[pl.ds(base, _BC)], h, si),
            (tails_hbm.at[pl.ds(base, _BC)], t, si),
            (wf_hbm.at[pl.ds(base, _BC)], f, si),
            (wb_hbm.at[pl.ds(base, _BC)], b, si))

  def idx_start(ci, bs):
    for a in idx_args(ci, bs):
      pltpu.async_copy(*a)

  def idx_wait(ci, bs):
    for a in idx_args(ci, bs):
      pltpu.make_async_copy(*a).wait()

  def gather_start(bs):
    h, t, tf, tb, sg = bs[0], bs[1], bs[6], bs[7], bs[9]
    pltpu.async_copy(tf_hbm.at[h], tf, sg)
    pltpu.async_copy(tb_hbm.at[t], tb, sg)

  def gather_wait(bs):
    h, t, tf, tb, sg = bs[0], bs[1], bs[6], bs[7], bs[9]
    pltpu.make_async_copy(tf_hbm.at[h], tf, sg).wait()
    pltpu.make_async_copy(tb_hbm.at[t], tb, sg).wait()

  def scatter_start(bs):
    sh, st, tf, tb, ss = bs[4], bs[5], bs[6], bs[7], bs[10]
    pltpu.async_copy(tf, acc.at[st], ss, add=True)
    pltpu.async_copy(tb, acc.at[sh], ss, add=True)

  def scatter_wait(bs):
    sh, st, tf, tb, ss = bs[4], bs[5], bs[6], bs[7], bs[10]
    pltpu.make_async_copy(tf, acc.at[st], ss).wait()
    pltpu.make_async_copy(tb, acc.at[sh], ss).wait()

  idx_start(0, bufs[0])
  idx_wait(0, bufs[0])
  gather_start(bufs[0])
  idx_start(1, bufs[1])

  def step(ci, cur, nxt):
    @pl.when(ci < _NCC)
    def _():
      gather_wait(cur)

      @pl.when(ci + 1 < _NCC)
      def _():
        @pl.when(ci >= 1)
        def _():
          scatter_wait(nxt)
        idx_wait(ci + 1, nxt)
        gather_start(nxt)

      h, t, f, b, sh, st, tf, tb = (cur[0], cur[1], cur[2], cur[3], cur[4],
                                    cur[5], cur[6], cur[7])
      _scale_rows2(tf, f, tb, b, _BC)
      _copy_idx(h, sh, _BC)
      _copy_idx(t, st, _BC)
      scatter_start(cur)

      @pl.when(ci + 2 < _NCC)
      def _():
        idx_start(ci + 2, cur)

  def pair(p, carry):
    step(2 * p, bufs[0], bufs[1])
    step(2 * p + 1, bufs[1], bufs[0])
    return carry

  lax.fori_loop(0, (_NCC + 1) // 2, pair, 0)
  scatter_wait(bufs[0])
  scatter_wait(bufs[1])
  plsc.subcore_barrier()
  pltpu.sync_copy(acc.at[pl.ds(r0, _RPT)], c_out.at[c, pl.ds(r0, _RPT)])


def _sc_body_q(e_hbm, dst_hbm, w_hbm, zeros_hbm, q_out,
               d0, w0, e0, d1, w1, e1, d2, w2, e2,
               acc, si0, si1, si2, ss0, ss1, ss2):
  c = lax.axis_index("c")
  s = lax.axis_index("s")
  wid = c * _NS + s

  r0 = s * _RPT
  pltpu.sync_copy(zeros_hbm, acc.at[pl.ds(r0, _RPT)])
  plsc.subcore_barrier()

  bufs = ((d0, w0, e0, si0, ss0),
          (d1, w1, e1, si1, ss1),
          (d2, w2, e2, si2, ss2))

  def in_args(ci, bs):
    d, w, e, si, _ = bs
    base = ci * _BQ
    return ((dst_hbm.at[pl.ds(base, _BQ)], d, si),
            (w_hbm.at[pl.ds(base, _BQ)], w, si),
            (e_hbm.at[pl.ds(base, _BQ)], e, si))

  def in_start(ci, bs):
    for a in in_args(ci, bs):
      pltpu.async_copy(*a)

  def in_wait(ci, bs):
    for a in in_args(ci, bs):
      pltpu.make_async_copy(*a).wait()

  def scatter_start(bs):
    d, _, e, _, ss = bs
    pltpu.async_copy(e, acc.at[d], ss, add=True)

  def scatter_wait(bs):
    d, _, e, _, ss = bs
    pltpu.make_async_copy(e, acc.at[d], ss).wait()

  in_start(wid, bufs[0])
  in_start(wid + _NW, bufs[1])
  in_start(wid + 2 * _NW, bufs[2])

  def step(i, cur, prv):
    ci = wid + i * _NW

    @pl.when(ci < _NCG)
    def _():
      in_wait(ci, cur)

      @pl.when(jnp.logical_and(i >= 1, ci + 2 * _NW < _NCG))
      def _():
        scatter_wait(prv)
        in_start(ci + 2 * _NW, prv)

      d, w, e = cur[0], cur[1], cur[2]
      _scale_rows(e, w, _BQ, bf16_round=True)
      scatter_start(cur)

  def trio(p, carry):
    step(3 * p, bufs[0], bufs[2])
    step(3 * p + 1, bufs[1], bufs[0])
    step(3 * p + 2, bufs[2], bufs[1])
    return carry

  lax.fori_loop(0, (_ITER + 2) // 3, trio, 0)
  scatter_wait(bufs[0])
  scatter_wait(bufs[1])
  scatter_wait(bufs[2])
  plsc.subcore_barrier()
  pltpu.sync_copy(acc.at[pl.ds(r0, _RPT)], q_out.at[c, pl.ds(r0, _RPT)])


def _sc_mesh():
  return plsc.VectorSubcoreMesh(core_axis_name="c", subcore_axis_name="s",
                                num_cores=_NC, num_subcores=_NS)


_OUT2 = jax.ShapeDtypeStruct((_NC, _NPAD, _D), jnp.float32)


@jax.jit
def _sc_accumulate(tf, tb, e, heads, tails, wf, wb, zeros):
  f_c = pl.kernel(
      _sc_body_c,
      out_type=_OUT2,
      mesh=_sc_mesh(),
      scratch_types=(
          [pltpu.VMEM((_BC,), jnp.int32)] * 2 +
          [pltpu.VMEM((_BC,), jnp.float32)] * 2 +
          [pltpu.VMEM((_BC,), jnp.int32)] * 2 +
          [pltpu.VMEM((_BC,), jnp.int32)] * 2 +
          [pltpu.VMEM((_BC,), jnp.float32)] * 2 +
          [pltpu.VMEM((_BC,), jnp.int32)] * 2 +
          [pltpu.VMEM((_BC, _D), jnp.float32)] * 4 +
          [pltpu.VMEM_SHARED((_NPAD, _D), jnp.float32)] +
          [pltpu.SemaphoreType.DMA] * 6
      ),
  )
  f_q = pl.kernel(
      _sc_body_q,
      out_type=_OUT2,
      mesh=_sc_mesh(),
      scratch_types=(
          ([pltpu.VMEM((_BQ,), jnp.int32),
            pltpu.VMEM((_BQ,), jnp.float32),
            pltpu.VMEM((_BQ, _D), jnp.float32)
            ] * 3) +
          [pltpu.VMEM_SHARED((_NPAD, _D), jnp.float32)] +
          [pltpu.SemaphoreType.DMA] * 6
      ),
  )
  C2 = f_c(tf, tb, heads, tails, wf, wb, zeros)
  Qf2 = f_q(e, tails, wf, zeros)
  Qb2 = f_q(e, heads, wb, zeros)
  return C2, Qf2, Qb2


def _pre_body(h_ref, wf_ref, bf_ref, wb_ref, bb_ref, tf_ref, tb_ref):
  h = h_ref[...]
  tf_ref[...] = lax.dot_general(h, wf_ref[...][:, :_D],
                                (((1,), (1,)), ((), ()))) + bf_ref[...]
  tb_ref[...] = lax.dot_general(h, wb_ref[...][:, :_D],
                                (((1,), (1,)), ((), ()))) + bb_ref[...]


@jax.jit
def _pre_tables(H, W_fwd, b_fwd, W_back, b_back):
  blk = 1000
  grid = (_N // blk,)
  return pl.pallas_call(
      _pre_body,
      grid=grid,
      in_specs=[
          pl.BlockSpec((blk, _D), lambda i: (i, 0)),
          pl.BlockSpec((_D, 2 * _D), lambda i: (0, 0)),
          pl.BlockSpec((1, _D), lambda i: (0, 0)),
          pl.BlockSpec((_D, 2 * _D), lambda i: (0, 0)),
          pl.BlockSpec((1, _D), lambda i: (0, 0)),
      ],
      out_specs=[
          pl.BlockSpec((blk, _D), lambda i: (i, 0)),
          pl.BlockSpec((blk, _D), lambda i: (i, 0)),
      ],
      out_shape=[jax.ShapeDtypeStruct((_N, _D), jnp.float32)] * 2,
  )(H, W_fwd, b_fwd, W_back, b_back)


def _post_body(c0_ref, c1_ref, qf0_ref, qf1_ref, qb0_ref, qb1_ref, h_ref,
               wf_ref, wb_ref, g_ref, b_ref, o_ref):
  agg = c0_ref[0] + c1_ref[0]
  qf = qf0_ref[0] + qf1_ref[0]
  qb = qb0_ref[0] + qb1_ref[0]
  w2f = wf_ref[...][:, _D:].astype(jnp.bfloat16).astype(jnp.float32)
  w2b = wb_ref[...][:, _D:].astype(jnp.bfloat16).astype(jnp.float32)
  agg = agg + lax.dot_general(qf, w2f, (((1,), (1,)), ((), ())),
                              precision=jax.lax.Precision.HIGHEST)
  agg = agg + lax.dot_general(qb, w2b, (((1,), (1,)), ((), ())),
                              precision=jax.lax.Precision.HIGHEST)
  agg = agg / jnp.sum(agg, axis=1, keepdims=True)
  x = jnp.where(agg >= 0, agg, 0.01 * agg) + h_ref[...]
  mean = jnp.mean(x, axis=1, keepdims=True)
  xc = x - mean
  var = jnp.mean(xc * xc, axis=1, keepdims=True)
  o_ref[...] = xc * jax.lax.rsqrt(var + 1e-5) * g_ref[...] + b_ref[...]


@jax.jit
def _post_combine(C2, Qf2, Qb2, H, W_fwd, W_back, gamma, beta):
  blk = 1000
  grid = (_N // blk,)
  spec0 = pl.BlockSpec((1, blk, _D), lambda i: (0, i, 0))
  spec1 = pl.BlockSpec((1, blk, _D), lambda i: (1, i, 0))
  rspec = pl.BlockSpec((blk, _D), lambda i: (i, 0))
  wspec = pl.BlockSpec((_D, 2 * _D), lambda i: (0, 0))
  vspec = pl.BlockSpec((1, _D), lambda i: (0, 0))
  return pl.pallas_call(
      _post_body,
      grid=grid,
      in_specs=[spec0, spec1, spec0, spec1, spec0, spec1, rspec,
                wspec, wspec, vspec, vspec],
      out_specs=rspec,
      out_shape=jax.ShapeDtypeStruct((_N, _D), jnp.float32),
  )(C2, C2, Qf2, Qf2, Qb2, Qb2, H, W_fwd, W_back, gamma, beta)


def kernel(H, E, ht, queries, influence_weights, W_fwd, b_fwd, W_back, b_back,
           ln_gamma, ln_beta):
  del queries
  heads = ht[:, 0]
  tails = ht[:, 1]
  wf = influence_weights[:_M, 0]
  wb = influence_weights[_M:, 0]

  Tf, Tb = _pre_tables(H, W_fwd, b_fwd.reshape(1, _D), W_back,
                       b_back.reshape(1, _D))
  zeros = jnp.zeros((_RPT, _D), jnp.float32)

  C2, Qf2, Qb2 = _sc_accumulate(Tf, Tb, E, heads, tails, wf, wb, zeros)

  return _post_combine(C2, Qf2, Qb2, H, W_fwd, W_back,
                       ln_gamma.reshape(1, _D), ln_beta.reshape(1, _D))


# speedup vs baseline: 1.7263x; 1.0059x over previous
import functools

import jax
import jax.numpy as jnp
from jax import lax
from jax.experimental import pallas as pl
from jax.experimental.pallas import tpu as pltpu
from jax.experimental.pallas import tpu_sc as plsc

_N = 10000
_NPAD = 10112
_M = 320000
_D = 128
_NC = 2
_NS = 16
_NW = _NC * _NS
_L = 16
_BC = 80
_EPT = _M // _NW
_NCC = _EPT // _BC
_BQ = 128
_NCG = _M // _BQ
_ITER = -(-_NCG // _NW)
_RPT = _NPAD // _NS

_GDN = jax.lax.GatherDimensionNumbers(
    offset_dims=(), collapsed_slice_dims=(0,), start_index_map=(0,))


def _lane_bcast(v16, j):
  idx = jnp.full((_L,), j, jnp.int32)
  return jax.lax.gather(v16, idx[:, None], _GDN, slice_sizes=(1,),
                        mode=jax.lax.GatherScatterMode.PROMISE_IN_BOUNDS)


def _bf16_round(v):
  c = v * 65537.0
  return c - (c - v)


def _scale_group(buf, w_v, g, bf16_round):
  w16 = w_v[pl.ds(g * _L, _L)]
  for j in range(_L):
    r = g * _L + j
    wb = _lane_bcast(w16, j)
    for k in range(_D // _L):
      sl = pl.ds(k * _L, _L)
      x = buf[r, sl]
      if bf16_round:
        x = _bf16_round(x)
      buf[r, sl] = x * wb


def _scale_rows(buf, w_v, nb, bf16_round=False):
  def body(g, carry):
    _scale_group(buf, w_v, g, bf16_round)
    return carry
  lax.fori_loop(0, nb // _L, body, 0)


def _scale_rows2(buf_a, w_a, buf_b, w_b, nb):
  def body(g, carry):
    _scale_group(buf_a, w_a, g, False)
    _scale_group(buf_b, w_b, g, False)
    return carry
  lax.fori_loop(0, nb // _L, body, 0)


def _copy_idx(src, dst, nb):
  for g in range(nb // _L):
    sl = pl.ds(g * _L, _L)
    dst[sl] = src[sl]


def _sc_body_c(tf_hbm, tb_hbm, heads_hbm, tails_hbm, wf_hbm, wb_hbm,
               zeros_hbm, c_out,
               h0, t0, f0, b0, sh0, st0, h1, t1, f1, b1, sh1, st1,
               tf0, tb0, tf1, tb1,
               acc, si0, si1, sg0, sg1, ss0, ss1):
  c = lax.axis_index("c")
  s = lax.axis_index("s")
  wid = c * _NS + s

  r0 = s * _RPT
  pltpu.sync_copy(zeros_hbm, acc.at[pl.ds(r0, _RPT)])
  plsc.subcore_barrier()

  bufs = ((h0, t0, f0, b0, sh0, st0, tf0, tb0, si0, sg0, ss0),
          (h1, t1, f1, b1, sh1, st1, tf1, tb1, si1, sg1, ss1))

  tile_base = wid * _EPT

  def idx_args(ci, bs):
    h, t, f, b = bs[0], bs[1], bs[2], bs[3]
    si = bs[8]
    base = tile_base + ci * _BC
    return ((heads_hbm.at[pl.ds(base, _BC)], h, si),
            (tails_hbm.at[pl.ds(base, _BC)], t, si),
            (wf_hbm.at[pl.ds(base, _BC)], f, si),
            (wb_hbm.at[pl.ds(base, _BC)], b, si))

  def idx_start(ci, bs):
    for a in idx_args(ci, bs):
      pltpu.async_copy(*a)

  def idx_wait(ci, bs):
    for a in idx_args(ci, bs):
      pltpu.make_async_copy(*a).wait()

  def gather_start(bs):
    h, t, tf, tb, sg = bs[0], bs[1], bs[6], bs[7], bs[9]
    pltpu.async_copy(tf_hbm.at[h], tf, sg)
    pltpu.async_copy(tb_hbm.at[t], tb, sg)

  def gather_wait(bs):
    h, t, tf, tb, sg = bs[0], bs[1], bs[6], bs[7], bs[9]
    pltpu.make_async_copy(tf_hbm.at[h], tf, sg).wait()
    pltpu.make_async_copy(tb_hbm.at[t], tb, sg).wait()

  def scatter_start(bs):
    sh, st, tf, tb, ss = bs[4], bs[5], bs[6], bs[7], bs[10]
    pltpu.async_copy(tf, acc.at[st], ss, add=True)
    pltpu.async_copy(tb, acc.at[sh], ss, add=True)

  def scatter_wait(bs):
    sh, st, tf, tb, ss = bs[4], bs[5], bs[6], bs[7], bs[10]
    pltpu.make_async_copy(tf, acc.at[st], ss).wait()
    pltpu.make_async_copy(tb, acc.at[sh], ss).wait()

  idx_start(0, bufs[0])
  idx_wait(0, bufs[0])
  gather_start(bufs[0])
  idx_start(1, bufs[1])

  def step(ci, cur, nxt):
    @pl.when(ci < _NCC)
    def _():
      gather_wait(cur)

      @pl.when(ci + 1 < _NCC)
      def _():
        @pl.when(ci >= 1)
        def _():
          scatter_wait(nxt)
        idx_wait(ci + 1, nxt)
        gather_start(nxt)

      h, t, f, b, sh, st, tf, tb = (cur[0], cur[1], cur[2], cur[3], cur[4],
                                    cur[5], cur[6], cur[7])
      _scale_rows2(tf, f, tb, b, _BC)
      _copy_idx(h, sh, _BC)
      _copy_idx(t, st, _BC)
      scatter_start(cur)

      @pl.when(ci + 2 < _NCC)
      def _():
        idx_start(ci + 2, cur)

  def pair(p, carry):
    step(2 * p, bufs[0], bufs[1])
    step(2 * p + 1, bufs[1], bufs[0])
    return carry

  lax.fori_loop(0, (_NCC + 1) // 2, pair, 0)
  scatter_wait(bufs[0])
  scatter_wait(bufs[1])
  plsc.subcore_barrier()
  pltpu.sync_copy(acc.at[pl.ds(r0, _RPT)], c_out.at[c, pl.ds(r0, _RPT)])


def _sc_body_q(e_hbm, tails_hbm, heads_hbm, wf_hbm, wb_hbm, zeros_hbm,
               qf_out, qb_out,
               d0, w0, e0, d1, w1, e1, d2, w2, e2,
               acc, si0, si1, si2, ss0, ss1, ss2):
  c = lax.axis_index("c")
  s = lax.axis_index("s")
  wid = c * _NS + s
  r0 = s * _RPT

  bufs = ((d0, w0, e0, si0, ss0),
          (d1, w1, e1, si1, ss1),
          (d2, w2, e2, si2, ss2))

  def phase(dst_hbm, w_hbm, q_out):
    pltpu.sync_copy(zeros_hbm, acc.at[pl.ds(r0, _RPT)])
    plsc.subcore_barrier()

    def in_args(ci, bs):
      d, w, e, si, _ = bs
      base = ci * _BQ
      return ((dst_hbm.at[pl.ds(base, _BQ)], d, si),
              (w_hbm.at[pl.ds(base, _BQ)], w, si),
              (e_hbm.at[pl.ds(base, _BQ)], e, si))

    def in_start(ci, bs):
      for a in in_args(ci, bs):
        pltpu.async_copy(*a)

    def in_wait(ci, bs):
      for a in in_args(ci, bs):
        pltpu.make_async_copy(*a).wait()

    def scatter_start(bs):
      d, _, e, _, ss = bs
      pltpu.async_copy(e, acc.at[d], ss, add=True)

    def scatter_wait(bs):
      d, _, e, _, ss = bs
      pltpu.make_async_copy(e, acc.at[d], ss).wait()

    in_start(wid, bufs[0])
    in_start(wid + _NW, bufs[1])
    in_start(wid + 2 * _NW, bufs[2])

    def step(i, cur, prv):
      ci = wid + i * _NW

      @pl.when(ci < _NCG)
      def _():
        in_wait(ci, cur)

        @pl.when(jnp.logical_and(i >= 1, ci + 2 * _NW < _NCG))
        def _():
          scatter_wait(prv)
          in_start(ci + 2 * _NW, prv)

        d, w, e = cur[0], cur[1], cur[2]
        _scale_rows(e, w, _BQ, bf16_round=True)
        scatter_start(cur)

    def trio(p, carry):
      step(3 * p, bufs[0], bufs[2])
      step(3 * p + 1, bufs[1], bufs[0])
      step(3 * p + 2, bufs[2], bufs[1])
      return carry

    lax.fori_loop(0, (_ITER + 2) // 3, trio, 0)
    scatter_wait(bufs[0])
    scatter_wait(bufs[1])
    scatter_wait(bufs[2])
    plsc.subcore_barrier()
    pltpu.sync_copy(acc.at[pl.ds(r0, _RPT)], q_out.at[c, pl.ds(r0, _RPT)])
    plsc.subcore_barrier()

  phase(tails_hbm, wf_hbm, qf_out)
  phase(heads_hbm, wb_hbm, qb_out)


def _sc_mesh():
  return plsc.VectorSubcoreMesh(core_axis_name="c", subcore_axis_name="s",
                                num_cores=_NC, num_subcores=_NS)


_OUT2 = jax.ShapeDtypeStruct((_NC, _NPAD, _D), jnp.float32)


@jax.jit
def _sc_accumulate(tf, tb, e, heads, tails, wf, wb, zeros):
  f_c = pl.kernel(
      _sc_body_c,
      out_type=_OUT2,
      mesh=_sc_mesh(),
      scratch_types=(
          [pltpu.VMEM((_BC,), jnp.int32)] * 2 +
          [pltpu.VMEM((_BC,), jnp.float32)] * 2 +
          [pltpu.VMEM((_BC,), jnp.int32)] * 2 +
          [pltpu.VMEM((_BC,), jnp.int32)] * 2 +
          [pltpu.VMEM((_BC,), jnp.float32)] * 2 +
          [pltpu.VMEM((_BC,), jnp.int32)] * 2 +
          [pltpu.VMEM((_BC, _D), jnp.float32)] * 4 +
          [pltpu.VMEM_SHARED((_NPAD, _D), jnp.float32)] +
          [pltpu.SemaphoreType.DMA] * 6
      ),
  )
  f_q = pl.kernel(
      _sc_body_q,
      out_type=[_OUT2] * 2,
      mesh=_sc_mesh(),
      scratch_types=(
          ([pltpu.VMEM((_BQ,), jnp.int32),
            pltpu.VMEM((_BQ,), jnp.float32),
            pltpu.VMEM((_BQ, _D), jnp.float32)
            ] * 3) +
          [pltpu.VMEM_SHARED((_NPAD, _D), jnp.float32)] +
          [pltpu.SemaphoreType.DMA] * 6
      ),
  )
  C2 = f_c(tf, tb, heads, tails, wf, wb, zeros)
  Qf2, Qb2 = f_q(e, tails, heads, wf, wb, zeros)
  return C2, Qf2, Qb2


def _pre_body(h_ref, wf_ref, bf_ref, wb_ref, bb_ref, tf_ref, tb_ref):
  h = h_ref[...]
  tf_ref[...] = lax.dot_general(h, wf_ref[...][:, :_D],
                                (((1,), (1,)), ((), ()))) + bf_ref[...]
  tb_ref[...] = lax.dot_general(h, wb_ref[...][:, :_D],
                                (((1,), (1,)), ((), ()))) + bb_ref[...]


@jax.jit
def _pre_tables(H, W_fwd, b_fwd, W_back, b_back):
  blk = 1000
  grid = (_N // blk,)
  return pl.pallas_call(
      _pre_body,
      grid=grid,
      in_specs=[
          pl.BlockSpec((blk, _D), lambda i: (i, 0)),
          pl.BlockSpec((_D, 2 * _D), lambda i: (0, 0)),
          pl.BlockSpec((1, _D), lambda i: (0, 0)),
          pl.BlockSpec((_D, 2 * _D), lambda i: (0, 0)),
          pl.BlockSpec((1, _D), lambda i: (0, 0)),
      ],
      out_specs=[
          pl.BlockSpec((blk, _D), lambda i: (i, 0)),
          pl.BlockSpec((blk, _D), lambda i: (i, 0)),
      ],
      out_shape=[jax.ShapeDtypeStruct((_N, _D), jnp.float32)] * 2,
  )(H, W_fwd, b_fwd, W_back, b_back)


def _post_body(c0_ref, c1_ref, qf0_ref, qf1_ref, qb0_ref, qb1_ref, h_ref,
               wf_ref, wb_ref, g_ref, b_ref, o_ref):
  agg = c0_ref[0] + c1_ref[0]
  qf = qf0_ref[0] + qf1_ref[0]
  qb = qb0_ref[0] + qb1_ref[0]
  w2f = wf_ref[...][:, _D:].astype(jnp.bfloat16).astype(jnp.float32)
  w2b = wb_ref[...][:, _D:].astype(jnp.bfloat16).astype(jnp.float32)
  agg = agg + lax.dot_general(qf, w2f, (((1,), (1,)), ((), ())),
                              precision=jax.lax.Precision.HIGHEST)
  agg = agg + lax.dot_general(qb, w2b, (((1,), (1,)), ((), ())),
                              precision=jax.lax.Precision.HIGHEST)
  agg = agg / jnp.sum(agg, axis=1, keepdims=True)
  x = jnp.where(agg >= 0, agg, 0.01 * agg) + h_ref[...]
  mean = jnp.mean(x, axis=1, keepdims=True)
  xc = x - mean
  var = jnp.mean(xc * xc, axis=1, keepdims=True)
  o_ref[...] = xc * jax.lax.rsqrt(var + 1e-5) * g_ref[...] + b_ref[...]


@jax.jit
def _post_combine(C2, Qf2, Qb2, H, W_fwd, W_back, gamma, beta):
  blk = 1000
  grid = (_N // blk,)
  spec0 = pl.BlockSpec((1, blk, _D), lambda i: (0, i, 0))
  spec1 = pl.BlockSpec((1, blk, _D), lambda i: (1, i, 0))
  rspec = pl.BlockSpec((blk, _D), lambda i: (i, 0))
  wspec = pl.BlockSpec((_D, 2 * _D), lambda i: (0, 0))
  vspec = pl.BlockSpec((1, _D), lambda i: (0, 0))
  return pl.pallas_call(
      _post_body,
      grid=grid,
      in_specs=[spec0, spec1, spec0, spec1, spec0, spec1, rspec,
                wspec, wspec, vspec, vspec],
      out_specs=rspec,
      out_shape=jax.ShapeDtypeStruct((_N, _D), jnp.float32),
  )(C2, C2, Qf2, Qf2, Qb2, Qb2, H, W_fwd, W_back, gamma, beta)


def kernel(H, E, ht, queries, influence_weights, W_fwd, b_fwd, W_back, b_back,
           ln_gamma, ln_beta):
  del queries
  heads = ht[:, 0]
  tails = ht[:, 1]
  wf = influence_weights[:_M, 0]
  wb = influence_weights[_M:, 0]

  Tf, Tb = _pre_tables(H, W_fwd, b_fwd.reshape(1, _D), W_back,
                       b_back.reshape(1, _D))
  zeros = jnp.zeros((_RPT, _D), jnp.float32)

  C2, Qf2, Qb2 = _sc_accumulate(Tf, Tb, E, heads, tails, wf, wb, zeros)

  return _post_combine(C2, Qf2, Qb2, H, W_fwd, W_back,
                       ln_gamma.reshape(1, _D), ln_beta.reshape(1, _D))


# speedup vs baseline: 1.7467x; 1.0118x over previous
import functools

import jax
import jax.numpy as jnp
from jax import lax
from jax.experimental import pallas as pl
from jax.experimental.pallas import tpu as pltpu
from jax.experimental.pallas import tpu_sc as plsc

_N = 10000
_NPAD = 10112
_M = 320000
_D = 128
_NC = 2
_NS = 16
_NW = _NC * _NS
_L = 16
_BC = 80
_EPT = _M // _NW
_NCC = _EPT // _BC
_BQ = 128
_NCG = _M // _BQ
_ITER = -(-_NCG // _NW)
_RPT = _NPAD // _NS

_GDN = jax.lax.GatherDimensionNumbers(
    offset_dims=(), collapsed_slice_dims=(0,), start_index_map=(0,))


def _lane_bcast(v16, j):
  idx = jnp.full((_L,), j, jnp.int32)
  return jax.lax.gather(v16, idx[:, None], _GDN, slice_sizes=(1,),
                        mode=jax.lax.GatherScatterMode.PROMISE_IN_BOUNDS)


def _bf16_round(v):
  c = v * 65537.0
  return c - (c - v)


def _scale_group(buf, w_v, g, bf16_round):
  w16 = w_v[pl.ds(g * _L, _L)]
  for j in range(_L):
    r = g * _L + j
    wb = _lane_bcast(w16, j)
    for k in range(_D // _L):
      sl = pl.ds(k * _L, _L)
      x = buf[r, sl]
      if bf16_round:
        x = _bf16_round(x)
      buf[r, sl] = x * wb


def _scale_rows(buf, w_v, nb, bf16_round=False):
  def body(g, carry):
    _scale_group(buf, w_v, g, bf16_round)
    return carry
  lax.fori_loop(0, nb // _L, body, 0)


def _scale_rows2(buf_a, w_a, buf_b, w_b, nb):
  def body(g, carry):
    _scale_group(buf_a, w_a, g, False)
    _scale_group(buf_b, w_b, g, False)
    return carry
  lax.fori_loop(0, nb // _L, body, 0)


def _copy_idx(src, dst, nb):
  for g in range(nb // _L):
    sl = pl.ds(g * _L, _L)
    dst[sl] = src[sl]


def _sc_body_c(tf_hbm, tb_hbm, heads_hbm, tails_hbm, wf_hbm, wb_hbm,
               zeros_hbm, c_out,
               h0, t0, f0, b0, sh0, st0, h1, t1, f1, b1, sh1, st1,
               tf0, tb0, tf1, tb1,
               acc, si0, si1, sg0, sg1, ss0, ss1):
  c = lax.axis_index("c")
  s = lax.axis_index("s")
  wid = c * _NS + s

  r0 = s * _RPT
  pltpu.sync_copy(zeros_hbm, acc.at[pl.ds(r0, _RPT)])
  plsc.subcore_barrier()

  bufs = ((h0, t0, f0, b0, sh0, st0, tf0, tb0, si0, sg0, ss0),
          (h1, t1, f1, b1, sh1, st1, tf1, tb1, si1, sg1, ss1))

  tile_base = wid * _EPT

  def idx_args(ci, bs):
    h, t, f, b = bs[0], bs[1], bs[2], bs[3]
    si = bs[8]
    base = tile_base + ci * _BC
    return ((heads_hbm.at[pl.ds(base, _BC)], h, si),
            (tails_hbm.at[pl.ds(base, _BC)], t, si),
            (wf_hbm.at[pl.ds(base, _BC)], f, si),
            (wb_hbm.at[pl.ds(base, _BC)], b, si))

  def idx_start(ci, bs):
    for a in idx_args(ci, bs):
      pltpu.async_copy(*a)

  def idx_wait(ci, bs):
    for a in idx_args(ci, bs):
      pltpu.make_async_copy(*a).wait()

  def gather_start(bs):
    h, t, tf, tb, sg = bs[0], bs[1], bs[6], bs[7], bs[9]
    pltpu.async_copy(tf_hbm.at[h], tf, sg)
    pltpu.async_copy(tb_hbm.at[t], tb, sg)

  def gather_wait(bs):
    h, t, tf, tb, sg = bs[0], bs[1], bs[6], bs[7], bs[9]
    pltpu.make_async_copy(tf_hbm.at[h], tf, sg).wait()
    pltpu.make_async_copy(tb_hbm.at[t], tb, sg).wait()

  def scatter_start(bs):
    sh, st, tf, tb, ss = bs[4], bs[5], bs[6], bs[7], bs[10]
    pltpu.async_copy(tf, acc.at[st], ss, add=True)
    pltpu.async_copy(tb, acc.at[sh], ss, add=True)

  def scatter_wait(bs):
    sh, st, tf, tb, ss = bs[4], bs[5], bs[6], bs[7], bs[10]
    pltpu.make_async_copy(tf, acc.at[st], ss).wait()
    pltpu.make_async_copy(tb, acc.at[sh], ss).wait()

  idx_start(0, bufs[0])
  idx_wait(0, bufs[0])
  gather_start(bufs[0])
  idx_start(1, bufs[1])

  def step(ci, cur, nxt):
    @pl.when(ci < _NCC)
    def _():
      @pl.when(ci + 1 < _NCC)
      def _():
        @pl.when(ci >= 1)
        def _():
          scatter_wait(nxt)
        idx_wait(ci + 1, nxt)
        gather_start(nxt)

      gather_wait(cur)

      h, t, f, b, sh, st, tf, tb = (cur[0], cur[1], cur[2], cur[3], cur[4],
                                    cur[5], cur[6], cur[7])
      _scale_rows2(tf, f, tb, b, _BC)
      _copy_idx(h, sh, _BC)
      _copy_idx(t, st, _BC)
      scatter_start(cur)

      @pl.when(ci + 2 < _NCC)
      def _():
        idx_start(ci + 2, cur)

  def pair(p, carry):
    step(2 * p, bufs[0], bufs[1])
    step(2 * p + 1, bufs[1], bufs[0])
    return carry

  lax.fori_loop(0, (_NCC + 1) // 2, pair, 0)
  scatter_wait(bufs[0])
  scatter_wait(bufs[1])
  plsc.subcore_barrier()
  pltpu.sync_copy(acc.at[pl.ds(r0, _RPT)], c_out.at[c, pl.ds(r0, _RPT)])


def _sc_body_q(e_hbm, tails_hbm, heads_hbm, wf_hbm, wb_hbm, zeros_hbm,
               qf_out, qb_out,
               d0, w0, e0, d1, w1, e1, d2, w2, e2,
               acc, si0, si1, si2, ss0, ss1, ss2):
  c = lax.axis_index("c")
  s = lax.axis_index("s")
  wid = c * _NS + s
  r0 = s * _RPT

  bufs = ((d0, w0, e0, si0, ss0),
          (d1, w1, e1, si1, ss1),
          (d2, w2, e2, si2, ss2))

  def phase(dst_hbm, w_hbm, q_out):
    pltpu.sync_copy(zeros_hbm, acc.at[pl.ds(r0, _RPT)])
    plsc.subcore_barrier()

    def in_args(ci, bs):
      d, w, e, si, _ = bs
      base = ci * _BQ
      return ((dst_hbm.at[pl.ds(base, _BQ)], d, si),
              (w_hbm.at[pl.ds(base, _BQ)], w, si),
              (e_hbm.at[pl.ds(base, _BQ)], e, si))

    def in_start(ci, bs):
      for a in in_args(ci, bs):
        pltpu.async_copy(*a)

    def in_wait(ci, bs):
      for a in in_args(ci, bs):
        pltpu.make_async_copy(*a).wait()

    def scatter_start(bs):
      d, _, e, _, ss = bs
      pltpu.async_copy(e, acc.at[d], ss, add=True)

    def scatter_wait(bs):
      d, _, e, _, ss = bs
      pltpu.make_async_copy(e, acc.at[d], ss).wait()

    in_start(wid, bufs[0])
    in_start(wid + _NW, bufs[1])
    in_start(wid + 2 * _NW, bufs[2])

    def step(i, cur, prv):
      ci = wid + i * _NW

      @pl.when(ci < _NCG)
      def _():
        in_wait(ci, cur)

        @pl.when(jnp.logical_and(i >= 1, ci + 2 * _NW < _NCG))
        def _():
          scatter_wait(prv)
          in_start(ci + 2 * _NW, prv)

        d, w, e = cur[0], cur[1], cur[2]
        _scale_rows(e, w, _BQ, bf16_round=True)
        scatter_start(cur)

    def trio(p, carry):
      step(3 * p, bufs[0], bufs[2])
      step(3 * p + 1, bufs[1], bufs[0])
      step(3 * p + 2, bufs[2], bufs[1])
      return carry

    lax.fori_loop(0, (_ITER + 2) // 3, trio, 0)
    scatter_wait(bufs[0])
    scatter_wait(bufs[1])
    scatter_wait(bufs[2])
    plsc.subcore_barrier()
    pltpu.sync_copy(acc.at[pl.ds(r0, _RPT)], q_out.at[c, pl.ds(r0, _RPT)])
    plsc.subcore_barrier()

  phase(tails_hbm, wf_hbm, qf_out)
  phase(heads_hbm, wb_hbm, qb_out)


def _sc_mesh():
  return plsc.VectorSubcoreMesh(core_axis_name="c", subcore_axis_name="s",
                                num_cores=_NC, num_subcores=_NS)


_OUT2 = jax.ShapeDtypeStruct((_NC, _NPAD, _D), jnp.float32)


@jax.jit
def _sc_accumulate(tf, tb, e, heads, tails, wf, wb, zeros):
  f_c = pl.kernel(
      _sc_body_c,
      out_type=_OUT2,
      mesh=_sc_mesh(),
      scratch_types=(
          [pltpu.VMEM((_BC,), jnp.int32)] * 2 +
          [pltpu.VMEM((_BC,), jnp.float32)] * 2 +
          [pltpu.VMEM((_BC,), jnp.int32)] * 2 +
          [pltpu.VMEM((_BC,), jnp.int32)] * 2 +
          [pltpu.VMEM((_BC,), jnp.float32)] * 2 +
          [pltpu.VMEM((_BC,), jnp.int32)] * 2 +
          [pltpu.VMEM((_BC, _D), jnp.float32)] * 4 +
          [pltpu.VMEM_SHARED((_NPAD, _D), jnp.float32)] +
          [pltpu.SemaphoreType.DMA] * 6
      ),
  )
  f_q = pl.kernel(
      _sc_body_q,
      out_type=[_OUT2] * 2,
      mesh=_sc_mesh(),
      scratch_types=(
          ([pltpu.VMEM((_BQ,), jnp.int32),
            pltpu.VMEM((_BQ,), jnp.float32),
            pltpu.VMEM((_BQ, _D), jnp.float32)
            ] * 3) +
          [pltpu.VMEM_SHARED((_NPAD, _D), jnp.float32)] +
          [pltpu.SemaphoreType.DMA] * 6
      ),
  )
  C2 = f_c(tf, tb, heads, tails, wf, wb, zeros)
  Qf2, Qb2 = f_q(e, tails, heads, wf, wb, zeros)
  return C2, Qf2, Qb2


def _pre_body(h_ref, wf_ref, bf_ref, wb_ref, bb_ref, tf_ref, tb_ref):
  h = h_ref[...]
  tf_ref[...] = lax.dot_general(h, wf_ref[...][:, :_D],
                                (((1,), (1,)), ((), ()))) + bf_ref[...]
  tb_ref[...] = lax.dot_general(h, wb_ref[...][:, :_D],
                                (((1,), (1,)), ((), ()))) + bb_ref[...]


@jax.jit
def _pre_tables(H, W_fwd, b_fwd, W_back, b_back):
  blk = 1000
  grid = (_N // blk,)
  return pl.pallas_call(
      _pre_body,
      grid=grid,
      in_specs=[
          pl.BlockSpec((blk, _D), lambda i: (i, 0)),
          pl.BlockSpec((_D, 2 * _D), lambda i: (0, 0)),
          pl.BlockSpec((1, _D), lambda i: (0, 0)),
          pl.BlockSpec((_D, 2 * _D), lambda i: (0, 0)),
          pl.BlockSpec((1, _D), lambda i: (0, 0)),
      ],
      out_specs=[
          pl.BlockSpec((blk, _D), lambda i: (i, 0)),
          pl.BlockSpec((blk, _D), lambda i: (i, 0)),
      ],
      out_shape=[jax.ShapeDtypeStruct((_N, _D), jnp.float32)] * 2,
  )(H, W_fwd, b_fwd, W_back, b_back)


def _post_body(c0_ref, c1_ref, qf0_ref, qf1_ref, qb0_ref, qb1_ref, h_ref,
               wf_ref, wb_ref, g_ref, b_ref, o_ref):
  agg = c0_ref[0] + c1_ref[0]
  qf = qf0_ref[0] + qf1_ref[0]
  qb = qb0_ref[0] + qb1_ref[0]
  w2f = wf_ref[...][:, _D:].astype(jnp.bfloat16).astype(jnp.float32)
  w2b = wb_ref[...][:, _D:].astype(jnp.bfloat16).astype(jnp.float32)
  agg = agg + lax.dot_general(qf, w2f, (((1,), (1,)), ((), ())),
                              precision=jax.lax.Precision.HIGHEST)
  agg = agg + lax.dot_general(qb, w2b, (((1,), (1,)), ((), ())),
                              precision=jax.lax.Precision.HIGHEST)
  agg = agg / jnp.sum(agg, axis=1, keepdims=True)
  x = jnp.where(agg >= 0, agg, 0.01 * agg) + h_ref[...]
  mean = jnp.mean(x, axis=1, keepdims=True)
  xc = x - mean
  var = jnp.mean(xc * xc, axis=1, keepdims=True)
  o_ref[...] = xc * jax.lax.rsqrt(var + 1e-5) * g_ref[...] + b_ref[...]


@jax.jit
def _post_combine(C2, Qf2, Qb2, H, W_fwd, W_back, gamma, beta):
  blk = 1000
  grid = (_N // blk,)
  spec0 = pl.BlockSpec((1, blk, _D), lambda i: (0, i, 0))
  spec1 = pl.BlockSpec((1, blk, _D), lambda i: (1, i, 0))
  rspec = pl.BlockSpec((blk, _D), lambda i: (i, 0))
  wspec = pl.BlockSpec((_D, 2 * _D), lambda i: (0, 0))
  vspec = pl.BlockSpec((1, _D), lambda i: (0, 0))
  return pl.pallas_call(
      _post_body,
      grid=grid,
      in_specs=[spec0, spec1, spec0, spec1, spec0, spec1, rspec,
                wspec, wspec, vspec, vspec],
      out_specs=rspec,
      out_shape=jax.ShapeDtypeStruct((_N, _D), jnp.float32),
  )(C2, C2, Qf2, Qf2, Qb2, Qb2, H, W_fwd, W_back, gamma, beta)


def kernel(H, E, ht, queries, influence_weights, W_fwd, b_fwd, W_back, b_back,
           ln_gamma, ln_beta):
  del queries
  heads = ht[:, 0]
  tails = ht[:, 1]
  wf = influence_weights[:_M, 0]
  wb = influence_weights[_M:, 0]

  Tf, Tb = _pre_tables(H, W_fwd, b_fwd.reshape(1, _D), W_back,
                       b_back.reshape(1, _D))
  zeros = jnp.zeros((_RPT, _D), jnp.float32)

  C2, Qf2, Qb2 = _sc_accumulate(Tf, Tb, E, heads, tails, wf, wb, zeros)

  return _post_combine(C2, Qf2, Qb2, H, W_fwd, W_back,
                       ln_gamma.reshape(1, _D), ln_beta.reshape(1, _D))


# speedup vs baseline: 1.7636x; 1.0097x over previous
import functools

import jax
import jax.numpy as jnp
from jax import lax
from jax.experimental import pallas as pl
from jax.experimental.pallas import tpu as pltpu
from jax.experimental.pallas import tpu_sc as plsc

_N = 10000
_NPAD = 10112
_M = 320000
_D = 128
_NC = 2
_NS = 16
_NW = _NC * _NS
_L = 16
_BC = 80
_EPT = _M // _NW
_NCC = _EPT // _BC
_BQ = 128
_NCG = _M // _BQ
_ITER = -(-_NCG // _NW)
_RPT = _NPAD // _NS

_GDN = jax.lax.GatherDimensionNumbers(
    offset_dims=(), collapsed_slice_dims=(0,), start_index_map=(0,))


def _lane_bcast(v16, j):
  idx = jnp.full((_L,), j, jnp.int32)
  return jax.lax.gather(v16, idx[:, None], _GDN, slice_sizes=(1,),
                        mode=jax.lax.GatherScatterMode.PROMISE_IN_BOUNDS)


def _bf16_round(v):
  c = v * 65537.0
  return c - (c - v)


def _scale_group(buf, w_v, g, bf16_round):
  w16 = w_v[pl.ds(g * _L, _L)]
  for j in range(_L):
    r = g * _L + j
    wb = _lane_bcast(w16, j)
    for k in range(_D // _L):
      sl = pl.ds(k * _L, _L)
      x = buf[r, sl]
      if bf16_round:
        x = _bf16_round(x)
      buf[r, sl] = x * wb


def _scale_rows(buf, w_v, nb, bf16_round=False):
  def body(g, carry):
    _scale_group(buf, w_v, g, bf16_round)
    return carry
  lax.fori_loop(0, nb // _L, body, 0)


def _scale_rows2(buf_a, w_a, buf_b, w_b, nb):
  def body(g, carry):
    _scale_group(buf_a, w_a, g, False)
    _scale_group(buf_b, w_b, g, False)
    return carry
  lax.fori_loop(0, nb // _L, body, 0)


def _copy_idx(src, dst, nb):
  for g in range(nb // _L):
    sl = pl.ds(g * _L, _L)
    dst[sl] = src[sl]


def _sc_body_c(tf_hbm, tb_hbm, heads_hbm, tails_hbm, wf_hbm, wb_hbm,
               zeros_hbm, c_out,
               h0, t0, f0, b0, sh0, st0, h1, t1, f1, b1, sh1, st1,
               tf0, tb0, tf1, tb1,
               acc, si0, si1, sg0, sg1, sgb0, sgb1, ss0, ss1):
  c = lax.axis_index("c")
  s = lax.axis_index("s")
  wid = c * _NS + s

  r0 = s * _RPT
  pltpu.sync_copy(zeros_hbm, acc.at[pl.ds(r0, _RPT)])
  plsc.subcore_barrier()

  bufs = ((h0, t0, f0, b0, sh0, st0, tf0, tb0, si0, sg0, ss0, sgb0),
          (h1, t1, f1, b1, sh1, st1, tf1, tb1, si1, sg1, ss1, sgb1))

  tile_base = wid * _EPT

  def idx_args(ci, bs):
    h, t, f, b = bs[0], bs[1], bs[2], bs[3]
    si = bs[8]
    base = tile_base + ci * _BC
    return ((heads_hbm.at[pl.ds(base, _BC)], h, si),
            (tails_hbm.at[pl.ds(base, _BC)], t, si),
            (wf_hbm.at[pl.ds(base, _BC)], f, si),
            (wb_hbm.at[pl.ds(base, _BC)], b, si))

  def idx_start(ci, bs):
    for a in idx_args(ci, bs):
      pltpu.async_copy(*a)

  def idx_wait(ci, bs):
    for a in idx_args(ci, bs):
      pltpu.make_async_copy(*a).wait()

  def gather_start(bs):
    h, t, tf, tb, sg, sgb = bs[0], bs[1], bs[6], bs[7], bs[9], bs[11]
    pltpu.async_copy(tf_hbm.at[h], tf, sg)
    pltpu.async_copy(tb_hbm.at[t], tb, sgb)

  def gather_wait_a(bs):
    h, tf, sg = bs[0], bs[6], bs[9]
    pltpu.make_async_copy(tf_hbm.at[h], tf, sg).wait()

  def gather_wait_b(bs):
    t, tb, sgb = bs[1], bs[7], bs[11]
    pltpu.make_async_copy(tb_hbm.at[t], tb, sgb).wait()

  def scatter_wait(bs):
    sh, st, tf, tb, ss = bs[4], bs[5], bs[6], bs[7], bs[10]
    pltpu.make_async_copy(tf, acc.at[st], ss).wait()
    pltpu.make_async_copy(tb, acc.at[sh], ss).wait()

  idx_start(0, bufs[0])
  idx_wait(0, bufs[0])
  gather_start(bufs[0])
  idx_start(1, bufs[1])

  def step(ci, cur, nxt):
    @pl.when(ci < _NCC)
    def _():
      @pl.when(ci + 1 < _NCC)
      def _():
        @pl.when(ci >= 1)
        def _():
          scatter_wait(nxt)
        idx_wait(ci + 1, nxt)
        gather_start(nxt)

      h, t, f, b, sh, st, tf, tb = (cur[0], cur[1], cur[2], cur[3], cur[4],
                                    cur[5], cur[6], cur[7])
      ss = cur[10]
      gather_wait_a(cur)
      _scale_rows(tf, f, _BC)
      _copy_idx(t, st, _BC)
      pltpu.async_copy(tf, acc.at[st], ss, add=True)
      gather_wait_b(cur)
      _scale_rows(tb, b, _BC)
      _copy_idx(h, sh, _BC)
      pltpu.async_copy(tb, acc.at[sh], ss, add=True)

      @pl.when(ci + 2 < _NCC)
      def _():
        idx_start(ci + 2, cur)

  def pair(p, carry):
    step(2 * p, bufs[0], bufs[1])
    step(2 * p + 1, bufs[1], bufs[0])
    return carry

  lax.fori_loop(0, (_NCC + 1) // 2, pair, 0)
  scatter_wait(bufs[0])
  scatter_wait(bufs[1])
  plsc.subcore_barrier()
  pltpu.sync_copy(acc.at[pl.ds(r0, _RPT)], c_out.at[c, pl.ds(r0, _RPT)])


def _sc_body_q(e_hbm, tails_hbm, heads_hbm, wf_hbm, wb_hbm, zeros_hbm,
               qf_out, qb_out,
               d0, w0, e0, d1, w1, e1, d2, w2, e2,
               acc, si0, si1, si2, ss0, ss1, ss2):
  c = lax.axis_index("c")
  s = lax.axis_index("s")
  wid = c * _NS + s
  r0 = s * _RPT

  bufs = ((d0, w0, e0, si0, ss0),
          (d1, w1, e1, si1, ss1),
          (d2, w2, e2, si2, ss2))

  def phase(dst_hbm, w_hbm, q_out):
    pltpu.sync_copy(zeros_hbm, acc.at[pl.ds(r0, _RPT)])
    plsc.subcore_barrier()

    def in_args(ci, bs):
      d, w, e, si, _ = bs
      base = ci * _BQ
      return ((dst_hbm.at[pl.ds(base, _BQ)], d, si),
              (w_hbm.at[pl.ds(base, _BQ)], w, si),
              (e_hbm.at[pl.ds(base, _BQ)], e, si))

    def in_start(ci, bs):
      for a in in_args(ci, bs):
        pltpu.async_copy(*a)

    def in_wait(ci, bs):
      for a in in_args(ci, bs):
        pltpu.make_async_copy(*a).wait()

    def scatter_start(bs):
      d, _, e, _, ss = bs
      pltpu.async_copy(e, acc.at[d], ss, add=True)

    def scatter_wait(bs):
      d, _, e, _, ss = bs
      pltpu.make_async_copy(e, acc.at[d], ss).wait()

    in_start(wid, bufs[0])
    in_start(wid + _NW, bufs[1])
    in_start(wid + 2 * _NW, bufs[2])

    def step(i, cur, prv):
      ci = wid + i * _NW

      @pl.when(ci < _NCG)
      def _():
        in_wait(ci, cur)

        @pl.when(jnp.logical_and(i >= 1, ci + 2 * _NW < _NCG))
        def _():
          scatter_wait(prv)
          in_start(ci + 2 * _NW, prv)

        _scale_rows(cur[2], cur[1], _BQ, bf16_round=True)
        scatter_start(cur)

    def trio(p, carry):
      step(3 * p, bufs[0], bufs[2])
      step(3 * p + 1, bufs[1], bufs[0])
      step(3 * p + 2, bufs[2], bufs[1])
      return carry

    lax.fori_loop(0, (_ITER + 2) // 3, trio, 0)
    scatter_wait(bufs[0])
    scatter_wait(bufs[1])
    scatter_wait(bufs[2])
    plsc.subcore_barrier()
    pltpu.sync_copy(acc.at[pl.ds(r0, _RPT)], q_out.at[c, pl.ds(r0, _RPT)])
    plsc.subcore_barrier()

  phase(tails_hbm, wf_hbm, qf_out)
  phase(heads_hbm, wb_hbm, qb_out)


def _sc_mesh():
  return plsc.VectorSubcoreMesh(core_axis_name="c", subcore_axis_name="s",
                                num_cores=_NC, num_subcores=_NS)


_OUT2 = jax.ShapeDtypeStruct((_NC, _NPAD, _D), jnp.float32)


@jax.jit
def _sc_accumulate(tf, tb, e, heads, tails, wf, wb, zeros):
  f_c = pl.kernel(
      _sc_body_c,
      out_type=_OUT2,
      mesh=_sc_mesh(),
      scratch_types=(
          [pltpu.VMEM((_BC,), jnp.int32)] * 2 +
          [pltpu.VMEM((_BC,), jnp.float32)] * 2 +
          [pltpu.VMEM((_BC,), jnp.int32)] * 2 +
          [pltpu.VMEM((_BC,), jnp.int32)] * 2 +
          [pltpu.VMEM((_BC,), jnp.float32)] * 2 +
          [pltpu.VMEM((_BC,), jnp.int32)] * 2 +
          [pltpu.VMEM((_BC, _D), jnp.float32)] * 4 +
          [pltpu.VMEM_SHARED((_NPAD, _D), jnp.float32)] +
          [pltpu.SemaphoreType.DMA] * 8
      ),
  )
  f_q = pl.kernel(
      _sc_body_q,
      out_type=[_OUT2] * 2,
      mesh=_sc_mesh(),
      scratch_types=(
          ([pltpu.VMEM((_BQ,), jnp.int32),
            pltpu.VMEM((_BQ,), jnp.float32),
            pltpu.VMEM((_BQ, _D), jnp.float32)
            ] * 3) +
          [pltpu.VMEM_SHARED((_NPAD, _D), jnp.float32)] +
          [pltpu.SemaphoreType.DMA] * 6
      ),
  )
  C2 = f_c(tf, tb, heads, tails, wf, wb, zeros)
  Qf2, Qb2 = f_q(e, tails, heads, wf, wb, zeros)
  return C2, Qf2, Qb2


def _pre_body(h_ref, wf_ref, bf_ref, wb_ref, bb_ref, tf_ref, tb_ref):
  h = h_ref[...]
  tf_ref[...] = lax.dot_general(h, wf_ref[...][:, :_D],
                                (((1,), (1,)), ((), ()))) + bf_ref[...]
  tb_ref[...] = lax.dot_general(h, wb_ref[...][:, :_D],
                                (((1,), (1,)), ((), ()))) + bb_ref[...]


@jax.jit
def _pre_tables(H, W_fwd, b_fwd, W_back, b_back):
  blk = 1000
  grid = (_N // blk,)
  return pl.pallas_call(
      _pre_body,
      grid=grid,
      in_specs=[
          pl.BlockSpec((blk, _D), lambda i: (i, 0)),
          pl.BlockSpec((_D, 2 * _D), lambda i: (0, 0)),
          pl.BlockSpec((1, _D), lambda i: (0, 0)),
          pl.BlockSpec((_D, 2 * _D), lambda i: (0, 0)),
          pl.BlockSpec((1, _D), lambda i: (0, 0)),
      ],
      out_specs=[
          pl.BlockSpec((blk, _D), lambda i: (i, 0)),
          pl.BlockSpec((blk, _D), lambda i: (i, 0)),
      ],
      out_shape=[jax.ShapeDtypeStruct((_N, _D), jnp.float32)] * 2,
  )(H, W_fwd, b_fwd, W_back, b_back)


def _post_body(c0_ref, c1_ref, qf0_ref, qf1_ref, qb0_ref, qb1_ref, h_ref,
               wf_ref, wb_ref, g_ref, b_ref, o_ref):
  agg = c0_ref[0] + c1_ref[0]
  qf = qf0_ref[0] + qf1_ref[0]
  qb = qb0_ref[0] + qb1_ref[0]
  w2f = wf_ref[...][:, _D:].astype(jnp.bfloat16).astype(jnp.float32)
  w2b = wb_ref[...][:, _D:].astype(jnp.bfloat16).astype(jnp.float32)
  agg = agg + lax.dot_general(qf, w2f, (((1,), (1,)), ((), ())),
                              precision=jax.lax.Precision.HIGHEST)
  agg = agg + lax.dot_general(qb, w2b, (((1,), (1,)), ((), ())),
                              precision=jax.lax.Precision.HIGHEST)
  agg = agg / jnp.sum(agg, axis=1, keepdims=True)
  x = jnp.where(agg >= 0, agg, 0.01 * agg) + h_ref[...]
  mean = jnp.mean(x, axis=1, keepdims=True)
  xc = x - mean
  var = jnp.mean(xc * xc, axis=1, keepdims=True)
  o_ref[...] = xc * jax.lax.rsqrt(var + 1e-5) * g_ref[...] + b_ref[...]


@jax.jit
def _post_combine(C2, Qf2, Qb2, H, W_fwd, W_back, gamma, beta):
  blk = 1000
  grid = (_N // blk,)
  spec0 = pl.BlockSpec((1, blk, _D), lambda i: (0, i, 0))
  spec1 = pl.BlockSpec((1, blk, _D), lambda i: (1, i, 0))
  rspec = pl.BlockSpec((blk, _D), lambda i: (i, 0))
  wspec = pl.BlockSpec((_D, 2 * _D), lambda i: (0, 0))
  vspec = pl.BlockSpec((1, _D), lambda i: (0, 0))
  return pl.pallas_call(
      _post_body,
      grid=grid,
      in_specs=[spec0, spec1, spec0, spec1, spec0, spec1, rspec,
                wspec, wspec, vspec, vspec],
      out_specs=rspec,
      out_shape=jax.ShapeDtypeStruct((_N, _D), jnp.float32),
  )(C2, C2, Qf2, Qf2, Qb2, Qb2, H, W_fwd, W_back, gamma, beta)


def kernel(H, E, ht, queries, influence_weights, W_fwd, b_fwd, W_back, b_back,
           ln_gamma, ln_beta):
  del queries
  heads = ht[:, 0]
  tails = ht[:, 1]
  wf = influence_weights[:_M, 0]
  wb = influence_weights[_M:, 0]

  Tf, Tb = _pre_tables(H, W_fwd, b_fwd.reshape(1, _D), W_back,
                       b_back.reshape(1, _D))
  zeros = jnp.zeros((_RPT, _D), jnp.float32)

  C2, Qf2, Qb2 = _sc_accumulate(Tf, Tb, E, heads, tails, wf, wb, zeros)

  return _post_combine(C2, Qf2, Qb2, H, W_fwd, W_back,
                       ln_gamma.reshape(1, _D), ln_beta.reshape(1, _D))


# speedup vs baseline: 1.7665x; 1.0016x over previous
import jax
import jax.numpy as jnp
from jax import lax
from jax.experimental import pallas as pl
from jax.experimental.pallas import tpu as pltpu
from jax.experimental.pallas import tpu_sc as plsc

_N = 10000
_NPAD = 10112
_M = 320000
_D = 128
_NC = 2
_NS = 16
_NW = _NC * _NS
_L = 16
_BC = 80
_EPT = _M // _NW
_NCC = _EPT // _BC
_BQ = 128
_NCG = _M // _BQ
_ITER = -(-_NCG // _NW)
_RPT = _NPAD // _NS

_GDN = jax.lax.GatherDimensionNumbers(
    offset_dims=(), collapsed_slice_dims=(0,), start_index_map=(0,))


def _lane_bcast(v16, j):
  idx = jnp.full((_L,), j, jnp.int32)
  return jax.lax.gather(v16, idx[:, None], _GDN, slice_sizes=(1,),
                        mode=jax.lax.GatherScatterMode.PROMISE_IN_BOUNDS)


def _bf16_round(v):
  c = v * 65537.0
  return c - (c - v)


def _scale_group(buf, w_v, g, bf16_round):
  w16 = w_v[pl.ds(g * _L, _L)]
  for j in range(_L):
    r = g * _L + j
    wb = _lane_bcast(w16, j)
    for k in range(_D // _L):
      sl = pl.ds(k * _L, _L)
      x = buf[r, sl]
      if bf16_round:
        x = _bf16_round(x)
      buf[r, sl] = x * wb


def _scale_rows(buf, w_v, nb, bf16_round=False):
  def body(g, carry):
    _scale_group(buf, w_v, g, bf16_round)
    return carry
  lax.fori_loop(0, nb // _L, body, 0)


def _copy_idx(src, dst, nb):
  for g in range(nb // _L):
    sl = pl.ds(g * _L, _L)
    dst[sl] = src[sl]


def _sc_body_c(tf_hbm, tb_hbm, heads_hbm, tails_hbm, wf_hbm, wb_hbm,
               zeros_hbm, c_out,
               h0, t0, f0, b0, sh0, st0, h1, t1, f1, b1, sh1, st1,
               tf0, tb0, tf1, tb1,
               acc, si0, si1, sg0, sg1, sgb0, sgb1, ss0, ss1):
  c = lax.axis_index("c")
  s = lax.axis_index("s")
  wid = c * _NS + s

  r0 = s * _RPT
  pltpu.sync_copy(zeros_hbm, acc.at[pl.ds(r0, _RPT)])
  plsc.subcore_barrier()

  bufs = ((h0, t0, f0, b0, sh0, st0, tf0, tb0, si0, sg0, ss0, sgb0),
          (h1, t1, f1, b1, sh1, st1, tf1, tb1, si1, sg1, ss1, sgb1))

  tile_base = wid * _EPT

  def idx_args(ci, bs):
    h, t, f, b = bs[0], bs[1], bs[2], bs[3]
    si = bs[8]
    base = tile_base + ci * _BC
    return ((heads_hbm.at[pl.ds(base, _BC)], h, si),
            (tails_hbm.at[pl.ds(base, _BC)], t, si),
            (wf_hbm.at[pl.ds(base, _BC)], f, si),
            (wb_hbm.at[pl.ds(base, _BC)], b, si))

  def idx_start(ci, bs):
    for a in idx_args(ci, bs):
      pltpu.async_copy(*a)

  def idx_wait(ci, bs):
    for a in idx_args(ci, bs):
      pltpu.make_async_copy(*a).wait()

  def gather_start(bs):
    h, t, tf, tb, sg, sgb = bs[0], bs[1], bs[6], bs[7], bs[9], bs[11]
    pltpu.async_copy(tf_hbm.at[h], tf, sg)
    pltpu.async_copy(tb_hbm.at[t], tb, sgb)

  def gather_wait_a(bs):
    h, tf, sg = bs[0], bs[6], bs[9]
    pltpu.make_async_copy(tf_hbm.at[h], tf, sg).wait()

  def gather_wait_b(bs):
    t, tb, sgb = bs[1], bs[7], bs[11]
    pltpu.make_async_copy(tb_hbm.at[t], tb, sgb).wait()

  def scatter_wait(bs):
    sh, st, tf, tb, ss = bs[4], bs[5], bs[6], bs[7], bs[10]
    pltpu.make_async_copy(tf, acc.at[st], ss).wait()
    pltpu.make_async_copy(tb, acc.at[sh], ss).wait()

  idx_start(0, bufs[0])
  idx_wait(0, bufs[0])
  gather_start(bufs[0])
  idx_start(1, bufs[1])

  def step(ci, cur, nxt):
    @pl.when(ci < _NCC)
    def _():
      @pl.when(ci + 1 < _NCC)
      def _():
        @pl.when(ci >= 1)
        def _():
          scatter_wait(nxt)
        idx_wait(ci + 1, nxt)
        gather_start(nxt)

      h, t, f, b, sh, st, tf, tb = (cur[0], cur[1], cur[2], cur[3], cur[4],
                                    cur[5], cur[6], cur[7])
      ss = cur[10]
      gather_wait_a(cur)
      _scale_rows(tf, f, _BC)
      _copy_idx(t, st, _BC)
      pltpu.async_copy(tf, acc.at[st], ss, add=True)
      gather_wait_b(cur)
      _scale_rows(tb, b, _BC)
      _copy_idx(h, sh, _BC)
      pltpu.async_copy(tb, acc.at[sh], ss, add=True)

      @pl.when(ci + 2 < _NCC)
      def _():
        idx_start(ci + 2, cur)

  def pair(p, carry):
    step(2 * p, bufs[0], bufs[1])
    step(2 * p + 1, bufs[1], bufs[0])
    return carry

  lax.fori_loop(0, (_NCC + 1) // 2, pair, 0)
  scatter_wait(bufs[0])
  scatter_wait(bufs[1])
  plsc.subcore_barrier()
  pltpu.sync_copy(acc.at[pl.ds(r0, _RPT)], c_out.at[c, pl.ds(r0, _RPT)])


def _sc_body_q(e_hbm, tails_hbm, heads_hbm, wf_hbm, wb_hbm, zeros_hbm,
               qf_out, qb_out,
               d0, w0, e0, d1, w1, e1, d2, w2, e2,
               acc, si0, si1, si2, ss0, ss1, ss2):
  c = lax.axis_index("c")
  s = lax.axis_index("s")
  wid = c * _NS + s
  r0 = s * _RPT

  bufs = ((d0, w0, e0, si0, ss0),
          (d1, w1, e1, si1, ss1),
          (d2, w2, e2, si2, ss2))

  def phase(dst_hbm, w_hbm, q_out):
    pltpu.sync_copy(zeros_hbm, acc.at[pl.ds(r0, _RPT)])
    plsc.subcore_barrier()

    def in_args(ci, bs):
      d, w, e, si, _ = bs
      base = ci * _BQ
      return ((dst_hbm.at[pl.ds(base, _BQ)], d, si),
              (w_hbm.at[pl.ds(base, _BQ)], w, si),
              (e_hbm.at[pl.ds(base, _BQ)], e, si))

    def in_start(ci, bs):
      for a in in_args(ci, bs):
        pltpu.async_copy(*a)

    def in_wait(ci, bs):
      for a in in_args(ci, bs):
        pltpu.make_async_copy(*a).wait()

    def scatter_start(bs):
      d, _, e, _, ss = bs
      pltpu.async_copy(e, acc.at[d], ss, add=True)

    def scatter_wait(bs):
      d, _, e, _, ss = bs
      pltpu.make_async_copy(e, acc.at[d], ss).wait()

    in_start(wid, bufs[0])
    in_start(wid + _NW, bufs[1])
    in_start(wid + 2 * _NW, bufs[2])

    def step(i, cur, prv):
      ci = wid + i * _NW

      @pl.when(ci < _NCG)
      def _():
        in_wait(ci, cur)

        @pl.when(jnp.logical_and(i >= 1, ci + 2 * _NW < _NCG))
        def _():
          scatter_wait(prv)
          in_start(ci + 2 * _NW, prv)

        _scale_rows(cur[2], cur[1], _BQ, bf16_round=True)
        scatter_start(cur)

    def trio(p, carry):
      step(3 * p, bufs[0], bufs[2])
      step(3 * p + 1, bufs[1], bufs[0])
      step(3 * p + 2, bufs[2], bufs[1])
      return carry

    lax.fori_loop(0, (_ITER + 2) // 3, trio, 0)
    scatter_wait(bufs[0])
    scatter_wait(bufs[1])
    scatter_wait(bufs[2])
    plsc.subcore_barrier()
    pltpu.sync_copy(acc.at[pl.ds(r0, _RPT)], q_out.at[c, pl.ds(r0, _RPT)])
    plsc.subcore_barrier()

  phase(tails_hbm, wf_hbm, qf_out)
  phase(heads_hbm, wb_hbm, qb_out)


def _sc_mesh():
  return plsc.VectorSubcoreMesh(core_axis_name="c", subcore_axis_name="s",
                                num_cores=_NC, num_subcores=_NS)


_OUT2 = jax.ShapeDtypeStruct((_NC, _NPAD, _D), jnp.float32)


@jax.jit
def _sc_accumulate(tf, tb, e, heads, tails, wf, wb, zeros):
  f_c = pl.kernel(
      _sc_body_c,
      out_type=_OUT2,
      mesh=_sc_mesh(),
      scratch_types=(
          [pltpu.VMEM((_BC,), jnp.int32)] * 2 +
          [pltpu.VMEM((_BC,), jnp.float32)] * 2 +
          [pltpu.VMEM((_BC,), jnp.int32)] * 2 +
          [pltpu.VMEM((_BC,), jnp.int32)] * 2 +
          [pltpu.VMEM((_BC,), jnp.float32)] * 2 +
          [pltpu.VMEM((_BC,), jnp.int32)] * 2 +
          [pltpu.VMEM((_BC, _D), jnp.float32)] * 4 +
          [pltpu.VMEM_SHARED((_NPAD, _D), jnp.float32)] +
          [pltpu.SemaphoreType.DMA] * 8
      ),
  )
  f_q = pl.kernel(
      _sc_body_q,
      out_type=[_OUT2] * 2,
      mesh=_sc_mesh(),
      scratch_types=(
          ([pltpu.VMEM((_BQ,), jnp.int32),
            pltpu.VMEM((_BQ,), jnp.float32),
            pltpu.VMEM((_BQ, _D), jnp.float32)
            ] * 3) +
          [pltpu.VMEM_SHARED((_NPAD, _D), jnp.float32)] +
          [pltpu.SemaphoreType.DMA] * 6
      ),
  )
  C2 = f_c(tf, tb, heads, tails, wf, wb, zeros)
  Qf2, Qb2 = f_q(e, tails, heads, wf, wb, zeros)
  return C2, Qf2, Qb2


def _pre_body(h_ref, wf_ref, bf_ref, wb_ref, bb_ref, tf_ref, tb_ref):
  h = h_ref[...]
  tf_ref[...] = lax.dot_general(h, wf_ref[...][:, :_D],
                                (((1,), (1,)), ((), ()))) + bf_ref[...]
  tb_ref[...] = lax.dot_general(h, wb_ref[...][:, :_D],
                                (((1,), (1,)), ((), ()))) + bb_ref[...]


@jax.jit
def _pre_tables(H, W_fwd, b_fwd, W_back, b_back):
  blk = 1000
  grid = (_N // blk,)
  return pl.pallas_call(
      _pre_body,
      grid=grid,
      in_specs=[
          pl.BlockSpec((blk, _D), lambda i: (i, 0)),
          pl.BlockSpec((_D, 2 * _D), lambda i: (0, 0)),
          pl.BlockSpec((1, _D), lambda i: (0, 0)),
          pl.BlockSpec((_D, 2 * _D), lambda i: (0, 0)),
          pl.BlockSpec((1, _D), lambda i: (0, 0)),
      ],
      out_specs=[
          pl.BlockSpec((blk, _D), lambda i: (i, 0)),
          pl.BlockSpec((blk, _D), lambda i: (i, 0)),
      ],
      out_shape=[jax.ShapeDtypeStruct((_N, _D), jnp.float32)] * 2,
  )(H, W_fwd, b_fwd, W_back, b_back)


def _post_body(c0_ref, c1_ref, qf0_ref, qf1_ref, qb0_ref, qb1_ref, h_ref,
               wf_ref, wb_ref, g_ref, b_ref, o_ref):
  agg = c0_ref[0] + c1_ref[0]
  qf = qf0_ref[0] + qf1_ref[0]
  qb = qb0_ref[0] + qb1_ref[0]
  w2f = wf_ref[...][:, _D:].astype(jnp.bfloat16).astype(jnp.float32)
  w2b = wb_ref[...][:, _D:].astype(jnp.bfloat16).astype(jnp.float32)
  agg = agg + lax.dot_general(qf, w2f, (((1,), (1,)), ((), ())),
                              precision=jax.lax.Precision.HIGHEST)
  agg = agg + lax.dot_general(qb, w2b, (((1,), (1,)), ((), ())),
                              precision=jax.lax.Precision.HIGHEST)
  agg = agg / jnp.sum(agg, axis=1, keepdims=True)
  x = jnp.where(agg >= 0, agg, 0.01 * agg) + h_ref[...]
  mean = jnp.mean(x, axis=1, keepdims=True)
  xc = x - mean
  var = jnp.mean(xc * xc, axis=1, keepdims=True)
  o_ref[...] = xc * jax.lax.rsqrt(var + 1e-5) * g_ref[...] + b_ref[...]


@jax.jit
def _post_combine(C2, Qf2, Qb2, H, W_fwd, W_back, gamma, beta):
  blk = 1000
  grid = (_N // blk,)
  spec0 = pl.BlockSpec((1, blk, _D), lambda i: (0, i, 0))
  spec1 = pl.BlockSpec((1, blk, _D), lambda i: (1, i, 0))
  rspec = pl.BlockSpec((blk, _D), lambda i: (i, 0))
  wspec = pl.BlockSpec((_D, 2 * _D), lambda i: (0, 0))
  vspec = pl.BlockSpec((1, _D), lambda i: (0, 0))
  return pl.pallas_call(
      _post_body,
      grid=grid,
      in_specs=[spec0, spec1, spec0, spec1, spec0, spec1, rspec,
                wspec, wspec, vspec, vspec],
      out_specs=rspec,
      out_shape=jax.ShapeDtypeStruct((_N, _D), jnp.float32),
  )(C2, C2, Qf2, Qf2, Qb2, Qb2, H, W_fwd, W_back, gamma, beta)


def kernel(H, E, ht, queries, influence_weights, W_fwd, b_fwd, W_back, b_back,
           ln_gamma, ln_beta):
  del queries
  heads = ht[:, 0]
  tails = ht[:, 1]
  wf = influence_weights[:_M, 0]
  wb = influence_weights[_M:, 0]

  Tf, Tb = _pre_tables(H, W_fwd, b_fwd.reshape(1, _D), W_back,
                       b_back.reshape(1, _D))
  zeros = jnp.zeros((_RPT, _D), jnp.float32)

  C2, Qf2, Qb2 = _sc_accumulate(Tf, Tb, E, heads, tails, wf, wb, zeros)

  return _post_combine(C2, Qf2, Qb2, H, W_fwd, W_back,
                       ln_gamma.reshape(1, _D), ln_beta.reshape(1, _D))
